# Initial kernel scaffold; baseline (speedup 1.0000x reference)
import jax
import jax.numpy as jnp
from jax import lax
from jax.experimental import pallas as pl
from jax.experimental.pallas import tpu as pltpu
from jax.experimental.pallas import tpu_sc as plsc

_N = 10000
_E = 320000
_NC = 2
_NS = 16
_K = 80
_EPT = _E // (_NC * _NS)
_NBLK = _EPT // _K
_RCH = _N // _K
_ZJ = (_RCH + _NS - 1) // _NS


def _sc_mesh():
    return plsc.VectorSubcoreMesh(core_axis_name="c", subcore_axis_name="s")


def _deg_body(dst_hbm, ones_hbm, z1d_hbm, out_hbm, didx_v, ones_v, deg_sh):
    c = lax.axis_index("c")
    s = lax.axis_index("s")
    wid = c * _NS + s
    pltpu.sync_copy(ones_hbm, ones_v)
    pltpu.sync_copy(z1d_hbm.at[pl.ds(0, 624)], deg_sh.at[pl.ds(s * 624, 624)])

    @pl.when(s == 0)
    def _():
        pltpu.sync_copy(z1d_hbm.at[pl.ds(0, 16)], deg_sh.at[pl.ds(9984, 16)])

    plsc.subcore_barrier()

    def body(i, carry):
        off = wid * _EPT + i * _K
        pltpu.sync_copy(dst_hbm.at[pl.ds(off, _K)], didx_v)
        pltpu.sync_copy(ones_v, deg_sh.at[didx_v], add=True)
        return carry

    lax.fori_loop(0, _NBLK, body, 0)
    plsc.subcore_barrier()
    pltpu.sync_copy(deg_sh.at[pl.ds(s * 624, 624)],
                    out_hbm.at[c, pl.ds(s * 624, 624)])

    @pl.when(s == 0)
    def _():
        pltpu.sync_copy(deg_sh.at[pl.ds(9984, 16)],
                        out_hbm.at[c, pl.ds(9984, 16)])


_deg_call = pl.kernel(
    _deg_body,
    out_type=jax.ShapeDtypeStruct((_NC, _N), jnp.float32),
    mesh=_sc_mesh(),
    scratch_types=[
        pltpu.VMEM((_K,), jnp.int32),
        pltpu.VMEM((_K,), jnp.float32),
        pltpu.VMEM_SHARED((_N,), jnp.float32),
    ],
)


def _agg_body(m_hbm, src_hbm, dst_hbm, zrows_hbm, out_hbm,
              sidx_v, didx_v, rows_v, acc_sh, sem):
    c = lax.axis_index("c")
    s = lax.axis_index("s")
    wid = c * _NS + s

    def zbody(j, carry):
        ch = s + j * _NS

        @pl.when(ch < _RCH)
        def _():
            pltpu.sync_copy(zrows_hbm, acc_sh.at[pl.ds(ch * _K, _K)])

        return carry

    lax.fori_loop(0, _ZJ, zbody, 0)
    plsc.subcore_barrier()

    def ebody(i, carry):
        off = wid * _EPT + i * _K
        pltpu.sync_copy(src_hbm.at[pl.ds(off, _K)], sidx_v)
        pltpu.sync_copy(dst_hbm.at[pl.ds(off, _K)], didx_v)
        pltpu.async_copy(m_hbm.at[sidx_v], rows_v, sem).wait()
        pltpu.sync_copy(rows_v, acc_sh.at[didx_v], add=True)
        return carry

    lax.fori_loop(0, _NBLK, ebody, 0)
    plsc.subcore_barrier()

    def obody(j, carry):
        ch = s + j * _NS

        @pl.when(ch < _RCH)
        def _():
            pltpu.sync_copy(acc_sh.at[pl.ds(ch * _K, _K)],
                            out_hbm.at[c, pl.ds(ch * _K, _K)])

        return carry

    lax.fori_loop(0, _ZJ, obody, 0)


def _make_agg(d):
    return pl.kernel(
        _agg_body,
        out_type=jax.ShapeDtypeStruct((_NC, _N, d), jnp.float32),
        mesh=_sc_mesh(),
        scratch_types=[
            pltpu.VMEM((_K,), jnp.int32),
            pltpu.VMEM((_K,), jnp.int32),
            pltpu.VMEM((_K, d), jnp.float32),
            pltpu.VMEM_SHARED((_N, d), jnp.float32),
            pltpu.SemaphoreType.DMA,
        ],
    )


_agg128 = _make_agg(128)
_agg64 = _make_agg(64)


_R = 1000
_G = _N // _R


def _tcA_body(deg0_ref, deg1_ref, x_ref, w1_ref, m1p_ref, dinv_ref):
    deg = deg0_ref[...] + deg1_ref[...] + 1.0
    dinv = 1.0 / jnp.sqrt(deg)
    m1 = jnp.dot(x_ref[...], w1_ref[...], preferred_element_type=jnp.float32)
    m1p_ref[...] = m1 * dinv
    dinv_ref[...] = dinv


_tcA = pl.pallas_call(
    _tcA_body,
    grid=(_G,),
    in_specs=[
        pl.BlockSpec((_R, 1), lambda i: (i, 0)),
        pl.BlockSpec((_R, 1), lambda i: (i, 0)),
        pl.BlockSpec((_R, 128), lambda i: (i, 0)),
        pl.BlockSpec((128, 128), lambda i: (0, 0)),
    ],
    out_specs=[
        pl.BlockSpec((_R, 128), lambda i: (i, 0)),
        pl.BlockSpec((_R, 1), lambda i: (i, 0)),
    ],
    out_shape=[
        jax.ShapeDtypeStruct((_N, 128), jnp.float32),
        jax.ShapeDtypeStruct((_N, 1), jnp.float32),
    ],
)


def _tcB_body(a0_ref, a1_ref, m1p_ref, dinv_ref, b1_ref, w2_ref, m2p_ref):
    dinv = dinv_ref[...]
    pre = dinv * (a0_ref[...] + a1_ref[...] + m1p_ref[...]) + b1_ref[...]
    h1 = jnp.maximum(pre, 0.0)
    m2 = jnp.dot(h1, w2_ref[...], preferred_element_type=jnp.float32)
    m2p_ref[...] = m2 * dinv


_tcB = pl.pallas_call(
    _tcB_body,
    grid=(_G,),
    in_specs=[
        pl.BlockSpec((_R, 128), lambda i: (i, 0)),
        pl.BlockSpec((_R, 128), lambda i: (i, 0)),
        pl.BlockSpec((_R, 128), lambda i: (i, 0)),
        pl.BlockSpec((_R, 1), lambda i: (i, 0)),
        pl.BlockSpec((1, 128), lambda i: (0, 0)),
        pl.BlockSpec((128, 64), lambda i: (0, 0)),
    ],
    out_specs=pl.BlockSpec((_R, 64), lambda i: (i, 0)),
    out_shape=jax.ShapeDtypeStruct((_N, 64), jnp.float32),
)


def _tcC_body(a0_ref, a1_ref, m2p_ref, dinv_ref, b2_ref,
              fw1_ref, fb1_ref, fw2_ref, fb2_ref,
              asn_ref, pen_ref, s_ref):
    i = pl.program_id(0)
    h2 = dinv_ref[...] * (a0_ref[...] + a1_ref[...] + m2p_ref[...]) + b2_ref[...]
    t = jnp.tanh(jnp.dot(h2, fw1_ref[...], preferred_element_type=jnp.float32)
                 + fb1_ref[...])
    logits = jnp.dot(t, fw2_ref[...], preferred_element_type=jnp.float32) + fb2_ref[...]
    mx = jnp.max(logits, axis=1, keepdims=True)
    e = jnp.exp(logits - mx)
    asn = e / jnp.sum(e, axis=1, keepdims=True)
    asn_ref[...] = asn
    d = asn - 0.5
    s1 = jnp.sum(d)
    s2 = jnp.sum(d * d)

    @pl.when(i == 0)
    def _():
        s_ref[0] = s1
        s_ref[1] = s2

    @pl.when(i > 0)
    def _():
        s_ref[0] += s1
        s_ref[1] += s2

    @pl.when(i == pl.num_programs(0) - 1)
    def _():
        n = 2.0 * _N
        pen_ref[0, 0] = (s_ref[1] - s_ref[0] * s_ref[0] / n) / (n - 1.0)


_tcC = pl.pallas_call(
    _tcC_body,
    grid=(_G,),
    in_specs=[
        pl.BlockSpec((_R, 64), lambda i: (i, 0)),
        pl.BlockSpec((_R, 64), lambda i: (i, 0)),
        pl.BlockSpec((_R, 64), lambda i: (i, 0)),
        pl.BlockSpec((_R, 1), lambda i: (i, 0)),
        pl.BlockSpec((1, 64), lambda i: (0, 0)),
        pl.BlockSpec((64, 32), lambda i: (0, 0)),
        pl.BlockSpec((1, 32), lambda i: (0, 0)),
        pl.BlockSpec((32, 2), lambda i: (0, 0)),
        pl.BlockSpec((1, 2), lambda i: (0, 0)),
    ],
    out_specs=[
        pl.BlockSpec((_R, 2), lambda i: (i, 0)),
        pl.BlockSpec((1, 1), lambda i: (0, 0)),
    ],
    out_shape=[
        jax.ShapeDtypeStruct((_N, 2), jnp.float32),
        jax.ShapeDtypeStruct((1, 1), jnp.float32),
    ],
    scratch_shapes=[pltpu.SMEM((2,), jnp.float32)],
)


def kernel(x, edge_index, W1, b1, W2, b2, fc1_W, fc1_b, fc2_W, fc2_b):
    src = edge_index[0]
    dst = edge_index[1]
    ones_k = jnp.ones((_K,), jnp.float32)
    z1d = jnp.zeros((1024,), jnp.float32)
    z128 = jnp.zeros((_K, 128), jnp.float32)
    z64 = jnp.zeros((_K, 64), jnp.float32)

    degp = _deg_call(dst, ones_k, z1d)
    deg0 = degp[0].reshape(_N, 1)
    deg1 = degp[1].reshape(_N, 1)
    m1p, dinv = _tcA(deg0, deg1, x, W1)
    acc1 = _agg128(m1p, src, dst, z128)
    m2p = _tcB(acc1[0], acc1[1], m1p, dinv, b1.reshape(1, -1), W2)
    acc2 = _agg64(m2p, src, dst, z64)
    asn, pen = _tcC(acc2[0], acc2[1], m2p, dinv, b2.reshape(1, -1),
                    fc1_W, fc1_b.reshape(1, -1), fc2_W, fc2_b.reshape(1, -1))
    return asn, pen.reshape(())


# speedup vs baseline: 13.3639x; 13.3639x over previous
import jax
import jax.numpy as jnp
from jax import lax
from jax.experimental import pallas as pl
from jax.experimental.pallas import tpu as pltpu
from jax.experimental.pallas import tpu_sc as plsc

_N = 10000
_E = 320000
_NC = 2
_NS = 16
_K = 80
_EPT = _E // (_NC * _NS)
_NBLK = _EPT // _K
_RCH = _N // _K
_ZJ = (_RCH + _NS - 1) // _NS


def _sc_mesh():
    return plsc.VectorSubcoreMesh(core_axis_name="c", subcore_axis_name="s")


def _deg_body(dst_hbm, ones_hbm, z1d_hbm, out_hbm, didx_v, ones_v, stg_v, deg_sh):
    c = lax.axis_index("c")
    s = lax.axis_index("s")
    wid = c * _NS + s
    pltpu.sync_copy(ones_hbm, ones_v)
    pltpu.sync_copy(z1d_hbm.at[pl.ds(0, 640)], stg_v)
    pltpu.sync_copy(stg_v.at[pl.ds(0, 624)], deg_sh.at[pl.ds(s * 624, 624)])

    @pl.when(s == 0)
    def _():
        pltpu.sync_copy(stg_v.at[pl.ds(0, 16)], deg_sh.at[pl.ds(9984, 16)])

    plsc.subcore_barrier()

    def body(i, carry):
        off = wid * _EPT + i * _K
        pltpu.sync_copy(dst_hbm.at[pl.ds(off, _K)], didx_v)
        pltpu.sync_copy(ones_v, deg_sh.at[didx_v], add=True)
        return carry

    lax.fori_loop(0, _NBLK, body, 0)
    plsc.subcore_barrier()
    pltpu.sync_copy(deg_sh.at[pl.ds(s * 624, 624)], stg_v.at[pl.ds(0, 624)])
    pltpu.sync_copy(stg_v.at[pl.ds(0, 624)],
                    out_hbm.at[pl.ds(c * _N + s * 624, 624)])

    @pl.when(s == 0)
    def _():
        pltpu.sync_copy(deg_sh.at[pl.ds(9984, 16)], stg_v.at[pl.ds(624, 16)])
        pltpu.sync_copy(stg_v.at[pl.ds(624, 16)],
                        out_hbm.at[pl.ds(c * _N + 9984, 16)])


_deg_call = pl.kernel(
    _deg_body,
    out_type=jax.ShapeDtypeStruct((_NC * _N,), jnp.float32),
    mesh=_sc_mesh(),
    scratch_types=[
        pltpu.VMEM((_K,), jnp.int32),
        pltpu.VMEM((_K,), jnp.float32),
        pltpu.VMEM((640,), jnp.float32),
        pltpu.VMEM_SHARED((_N,), jnp.float32),
    ],
    compiler_params=pltpu.CompilerParams(use_tc_tiling_on_sc=False),
)


def _agg_body(m_hbm, src_hbm, dst_hbm, zrows_hbm, out_hbm,
              sidx_v, didx_v, rows_v, acc_sh, sem):
    c = lax.axis_index("c")
    s = lax.axis_index("s")
    wid = c * _NS + s

    pltpu.sync_copy(zrows_hbm, rows_v)

    def zbody(j, carry):
        ch = s + j * _NS

        @pl.when(ch < _RCH)
        def _():
            pltpu.sync_copy(rows_v, acc_sh.at[pl.ds(ch * _K, _K)])

        return carry

    lax.fori_loop(0, _ZJ, zbody, 0)
    plsc.subcore_barrier()

    def ebody(i, carry):
        off = wid * _EPT + i * _K
        pltpu.sync_copy(src_hbm.at[pl.ds(off, _K)], sidx_v)
        pltpu.sync_copy(dst_hbm.at[pl.ds(off, _K)], didx_v)
        pltpu.async_copy(m_hbm.at[sidx_v], rows_v, sem).wait()
        pltpu.sync_copy(rows_v, acc_sh.at[didx_v], add=True)
        return carry

    lax.fori_loop(0, _NBLK, ebody, 0)
    plsc.subcore_barrier()

    def obody(j, carry):
        ch = s + j * _NS

        @pl.when(ch < _RCH)
        def _():
            pltpu.sync_copy(acc_sh.at[pl.ds(ch * _K, _K)], rows_v)
            pltpu.sync_copy(rows_v, out_hbm.at[c, pl.ds(ch * _K, _K)])

        return carry

    lax.fori_loop(0, _ZJ, obody, 0)


def _make_agg(d):
    return pl.kernel(
        _agg_body,
        out_type=jax.ShapeDtypeStruct((_NC, _N, d), jnp.float32),
        mesh=_sc_mesh(),
        scratch_types=[
            pltpu.VMEM((_K,), jnp.int32),
            pltpu.VMEM((_K,), jnp.int32),
            pltpu.VMEM((_K, d), jnp.float32),
            pltpu.VMEM_SHARED((_N, d), jnp.float32),
            pltpu.SemaphoreType.DMA,
        ],
        compiler_params=pltpu.CompilerParams(use_tc_tiling_on_sc=False),
    )


_agg128 = _make_agg(128)
_agg64 = _make_agg(64)


_R = 1000
_G = _N // _R


def _tcA_body(deg0_ref, deg1_ref, x_ref, w1_ref, m1p_ref, dinv_ref):
    deg = deg0_ref[...] + deg1_ref[...] + 1.0
    dinv = 1.0 / jnp.sqrt(deg)
    m1 = jnp.dot(x_ref[...], w1_ref[...], preferred_element_type=jnp.float32)
    m1p_ref[...] = m1 * dinv
    dinv_ref[...] = dinv


_tcA = pl.pallas_call(
    _tcA_body,
    grid=(_G,),
    in_specs=[
        pl.BlockSpec((_R, 1), lambda i: (i, 0)),
        pl.BlockSpec((_R, 1), lambda i: (i, 0)),
        pl.BlockSpec((_R, 128), lambda i: (i, 0)),
        pl.BlockSpec((128, 128), lambda i: (0, 0)),
    ],
    out_specs=[
        pl.BlockSpec((_R, 128), lambda i: (i, 0)),
        pl.BlockSpec((_R, 1), lambda i: (i, 0)),
    ],
    out_shape=[
        jax.ShapeDtypeStruct((_N, 128), jnp.float32),
        jax.ShapeDtypeStruct((_N, 1), jnp.float32),
    ],
)


def _tcB_body(a0_ref, a1_ref, m1p_ref, dinv_ref, b1_ref, w2_ref, m2p_ref):
    dinv = dinv_ref[...]
    pre = dinv * (a0_ref[...] + a1_ref[...] + m1p_ref[...]) + b1_ref[...]
    h1 = jnp.maximum(pre, 0.0)
    m2 = jnp.dot(h1, w2_ref[...], preferred_element_type=jnp.float32)
    m2p_ref[...] = m2 * dinv


_tcB = pl.pallas_call(
    _tcB_body,
    grid=(_G,),
    in_specs=[
        pl.BlockSpec((_R, 128), lambda i: (i, 0)),
        pl.BlockSpec((_R, 128), lambda i: (i, 0)),
        pl.BlockSpec((_R, 128), lambda i: (i, 0)),
        pl.BlockSpec((_R, 1), lambda i: (i, 0)),
        pl.BlockSpec((1, 128), lambda i: (0, 0)),
        pl.BlockSpec((128, 64), lambda i: (0, 0)),
    ],
    out_specs=pl.BlockSpec((_R, 64), lambda i: (i, 0)),
    out_shape=jax.ShapeDtypeStruct((_N, 64), jnp.float32),
)


def _tcC_body(a0_ref, a1_ref, m2p_ref, dinv_ref, b2_ref,
              fw1_ref, fb1_ref, fw2_ref, fb2_ref,
              asn_ref, pen_ref, s_ref):
    i = pl.program_id(0)
    h2 = dinv_ref[...] * (a0_ref[...] + a1_ref[...] + m2p_ref[...]) + b2_ref[...]
    t = jnp.tanh(jnp.dot(h2, fw1_ref[...], preferred_element_type=jnp.float32)
                 + fb1_ref[...])
    logits = jnp.dot(t, fw2_ref[...], preferred_element_type=jnp.float32) + fb2_ref[...]
    mx = jnp.max(logits, axis=1, keepdims=True)
    e = jnp.exp(logits - mx)
    asn = e / jnp.sum(e, axis=1, keepdims=True)
    asn_ref[...] = asn
    d = asn - 0.5
    s1 = jnp.sum(d)
    s2 = jnp.sum(d * d)

    @pl.when(i == 0)
    def _():
        s_ref[0] = s1
        s_ref[1] = s2

    @pl.when(i > 0)
    def _():
        s_ref[0] += s1
        s_ref[1] += s2

    @pl.when(i == pl.num_programs(0) - 1)
    def _():
        n = 2.0 * _N
        var = (s_ref[1] - s_ref[0] * s_ref[0] / n) / (n - 1.0)
        pen_ref[...] = jnp.full((1, 1), var, dtype=jnp.float32)


_tcC = pl.pallas_call(
    _tcC_body,
    grid=(_G,),
    in_specs=[
        pl.BlockSpec((_R, 64), lambda i: (i, 0)),
        pl.BlockSpec((_R, 64), lambda i: (i, 0)),
        pl.BlockSpec((_R, 64), lambda i: (i, 0)),
        pl.BlockSpec((_R, 1), lambda i: (i, 0)),
        pl.BlockSpec((1, 64), lambda i: (0, 0)),
        pl.BlockSpec((64, 32), lambda i: (0, 0)),
        pl.BlockSpec((1, 32), lambda i: (0, 0)),
        pl.BlockSpec((32, 2), lambda i: (0, 0)),
        pl.BlockSpec((1, 2), lambda i: (0, 0)),
    ],
    out_specs=[
        pl.BlockSpec((_R, 2), lambda i: (i, 0)),
        pl.BlockSpec((1, 1), lambda i: (0, 0)),
    ],
    out_shape=[
        jax.ShapeDtypeStruct((_N, 2), jnp.float32),
        jax.ShapeDtypeStruct((1, 1), jnp.float32),
    ],
    scratch_shapes=[pltpu.SMEM((2,), jnp.float32)],
)


def kernel(x, edge_index, W1, b1, W2, b2, fc1_W, fc1_b, fc2_W, fc2_b):
    src = edge_index[0]
    dst = edge_index[1]
    ones_k = jnp.ones((_K,), jnp.float32)
    z1d = jnp.zeros((1024,), jnp.float32)
    z128 = jnp.zeros((_K, 128), jnp.float32)
    z64 = jnp.zeros((_K, 64), jnp.float32)

    degp = _deg_call(dst, ones_k, z1d)
    deg0 = degp[:_N].reshape(_N, 1)
    deg1 = degp[_N:].reshape(_N, 1)
    m1p, dinv = _tcA(deg0, deg1, x, W1)
    acc1 = _agg128(m1p, src, dst, z128)
    m2p = _tcB(acc1[0], acc1[1], m1p, dinv, b1.reshape(1, -1), W2)
    acc2 = _agg64(m2p, src, dst, z64)
    asn, pen = _tcC(acc2[0], acc2[1], m2p, dinv, b2.reshape(1, -1),
                    fc1_W, fc1_b.reshape(1, -1), fc2_W, fc2_b.reshape(1, -1))
    return asn, pen.reshape(())


# speedup vs baseline: 30.4269x; 2.2768x over previous
import jax
import jax.numpy as jnp
from jax import lax
from jax.experimental import pallas as pl
from jax.experimental.pallas import tpu as pltpu
from jax.experimental.pallas import tpu_sc as plsc

_N = 10000
_E = 320000
_NC = 2
_NS = 16
_K = 80
_EPT = _E // (_NC * _NS)
_NBLK = _EPT // _K
_RCH = _N // _K
_ZJ = (_RCH + _NS - 1) // _NS


def _sc_mesh():
    return plsc.VectorSubcoreMesh(core_axis_name="c", subcore_axis_name="s")


def _deg_body(dst2_hbm, ones_hbm, z1d_hbm, out_hbm, didx_v, ones_v, stg_v,
              deg_sh, sem_a, sem_b):
    c = lax.axis_index("c")
    s = lax.axis_index("s")
    wid = c * _NS + s
    cp = pltpu.async_copy(dst2_hbm.at[pl.ds(wid * _NBLK, _NBLK)], didx_v,
                          sem_a)
    pltpu.sync_copy(ones_hbm, ones_v)
    pltpu.sync_copy(z1d_hbm.at[pl.ds(0, 640)], stg_v)
    pltpu.sync_copy(stg_v.at[pl.ds(0, 624)], deg_sh.at[pl.ds(s * 624, 624)])

    @pl.when(s == 0)
    def _():
        pltpu.sync_copy(stg_v.at[pl.ds(0, 16)], deg_sh.at[pl.ds(9984, 16)])

    cp.wait()
    plsc.subcore_barrier()

    def _fire(i, sem):
        pltpu.async_copy(ones_v, deg_sh.at[didx_v.at[i]], sem, add=True)

    def _drain(i, sem):
        pltpu.make_async_copy(ones_v, deg_sh.at[didx_v.at[i]], sem).wait()

    _fire(0, sem_a)

    def body(j, carry):
        i0 = 2 * j
        i1 = 2 * j + 1
        i2 = 2 * j + 2

        @pl.when(i1 < _NBLK)
        def _():
            _fire(i1, sem_b)

        _drain(i0, sem_a)

        @pl.when(i2 < _NBLK)
        def _():
            _fire(i2, sem_a)

        @pl.when(i1 < _NBLK)
        def _():
            _drain(i1, sem_b)

        return carry

    lax.fori_loop(0, (_NBLK + 1) // 2, body, 0)
    plsc.subcore_barrier()
    pltpu.sync_copy(deg_sh.at[pl.ds(s * 624, 624)], stg_v.at[pl.ds(0, 624)])
    pltpu.sync_copy(stg_v.at[pl.ds(0, 624)],
                    out_hbm.at[pl.ds(c * _N + s * 624, 624)])

    @pl.when(s == 0)
    def _():
        pltpu.sync_copy(deg_sh.at[pl.ds(9984, 16)], stg_v.at[pl.ds(624, 16)])
        pltpu.sync_copy(stg_v.at[pl.ds(624, 16)],
                        out_hbm.at[pl.ds(c * _N + 9984, 16)])


_deg_call = pl.kernel(
    _deg_body,
    out_type=jax.ShapeDtypeStruct((_NC * _N,), jnp.float32),
    mesh=_sc_mesh(),
    scratch_types=[
        pltpu.VMEM((_NBLK, _K), jnp.int32),
        pltpu.VMEM((_K,), jnp.float32),
        pltpu.VMEM((640,), jnp.float32),
        pltpu.VMEM_SHARED((_N,), jnp.float32),
        pltpu.SemaphoreType.DMA,
        pltpu.SemaphoreType.DMA,
    ],
    compiler_params=pltpu.CompilerParams(use_tc_tiling_on_sc=False),
)


def _agg_body(m_hbm, src2_hbm, dst2_hbm, zrows_hbm, out_hbm,
              sidx_v, didx_v, rows0_v, rows1_v, acc_sh, sem_a, sem_b):
    c = lax.axis_index("c")
    s = lax.axis_index("s")
    wid = c * _NS + s

    cps = pltpu.async_copy(src2_hbm.at[pl.ds(wid * _NBLK, _NBLK)], sidx_v,
                           sem_a)
    cpd = pltpu.async_copy(dst2_hbm.at[pl.ds(wid * _NBLK, _NBLK)], didx_v,
                           sem_b)
    pltpu.sync_copy(zrows_hbm, rows0_v)

    def zbody(j, carry):
        ch = s + j * _NS

        @pl.when(ch < _RCH)
        def _():
            pltpu.sync_copy(rows0_v, acc_sh.at[pl.ds(ch * _K, _K)])

        return carry

    lax.fori_loop(0, _ZJ, zbody, 0)
    cps.wait()
    cpd.wait()
    plsc.subcore_barrier()

    def _gstart(i, rows, sem):
        pltpu.async_copy(m_hbm.at[sidx_v.at[i]], rows, sem)

    def _gwait(i, rows, sem):
        pltpu.make_async_copy(m_hbm.at[sidx_v.at[i]], rows, sem).wait()

    _gstart(0, rows0_v, sem_a)

    def ebody(j, carry):
        i0 = 2 * j
        i1 = 2 * j + 1
        i2 = 2 * j + 2

        @pl.when(i1 < _NBLK)
        def _():
            _gstart(i1, rows1_v, sem_b)

        _gwait(i0, rows0_v, sem_a)
        pltpu.sync_copy(rows0_v, acc_sh.at[didx_v.at[i0]], add=True)

        @pl.when(i2 < _NBLK)
        def _():
            _gstart(i2, rows0_v, sem_a)

        @pl.when(i1 < _NBLK)
        def _():
            _gwait(i1, rows1_v, sem_b)
            pltpu.sync_copy(rows1_v, acc_sh.at[didx_v.at[i1]], add=True)

        return carry

    lax.fori_loop(0, (_NBLK + 1) // 2, ebody, 0)
    plsc.subcore_barrier()

    def obody(j, carry):
        ch = s + j * _NS

        @pl.when(ch < _RCH)
        def _():
            pltpu.sync_copy(acc_sh.at[pl.ds(ch * _K, _K)], rows0_v)
            pltpu.sync_copy(rows0_v, out_hbm.at[c, pl.ds(ch * _K, _K)])

        return carry

    lax.fori_loop(0, _ZJ, obody, 0)


def _make_agg(d):
    return pl.kernel(
        _agg_body,
        out_type=jax.ShapeDtypeStruct((_NC, _N, d), jnp.float32),
        mesh=_sc_mesh(),
        scratch_types=[
            pltpu.VMEM((_NBLK, _K), jnp.int32),
            pltpu.VMEM((_NBLK, _K), jnp.int32),
            pltpu.VMEM((_K, d), jnp.float32),
            pltpu.VMEM((_K, d), jnp.float32),
            pltpu.VMEM_SHARED((_N, d), jnp.float32),
            pltpu.SemaphoreType.DMA,
            pltpu.SemaphoreType.DMA,
        ],
        compiler_params=pltpu.CompilerParams(use_tc_tiling_on_sc=False),
    )


_agg128 = _make_agg(128)
_agg64 = _make_agg(64)


_R = 1000
_G = _N // _R


def _tcA_body(deg0_ref, deg1_ref, x_ref, w1_ref, m1p_ref, dinv_ref):
    deg = deg0_ref[...] + deg1_ref[...] + 1.0
    dinv = 1.0 / jnp.sqrt(deg)
    m1 = jnp.dot(x_ref[...], w1_ref[...], preferred_element_type=jnp.float32)
    m1p_ref[...] = m1 * dinv
    dinv_ref[...] = dinv


_tcA = pl.pallas_call(
    _tcA_body,
    grid=(_G,),
    in_specs=[
        pl.BlockSpec((_R, 1), lambda i: (i, 0)),
        pl.BlockSpec((_R, 1), lambda i: (i, 0)),
        pl.BlockSpec((_R, 128), lambda i: (i, 0)),
        pl.BlockSpec((128, 128), lambda i: (0, 0)),
    ],
    out_specs=[
        pl.BlockSpec((_R, 128), lambda i: (i, 0)),
        pl.BlockSpec((_R, 1), lambda i: (i, 0)),
    ],
    out_shape=[
        jax.ShapeDtypeStruct((_N, 128), jnp.float32),
        jax.ShapeDtypeStruct((_N, 1), jnp.float32),
    ],
)


def _tcB_body(a0_ref, a1_ref, m1p_ref, dinv_ref, b1_ref, w2_ref, m2p_ref):
    dinv = dinv_ref[...]
    pre = dinv * (a0_ref[...] + a1_ref[...] + m1p_ref[...]) + b1_ref[...]
    h1 = jnp.maximum(pre, 0.0)
    m2 = jnp.dot(h1, w2_ref[...], preferred_element_type=jnp.float32)
    m2p_ref[...] = m2 * dinv


_tcB = pl.pallas_call(
    _tcB_body,
    grid=(_G,),
    in_specs=[
        pl.BlockSpec((_R, 128), lambda i: (i, 0)),
        pl.BlockSpec((_R, 128), lambda i: (i, 0)),
        pl.BlockSpec((_R, 128), lambda i: (i, 0)),
        pl.BlockSpec((_R, 1), lambda i: (i, 0)),
        pl.BlockSpec((1, 128), lambda i: (0, 0)),
        pl.BlockSpec((128, 64), lambda i: (0, 0)),
    ],
    out_specs=pl.BlockSpec((_R, 64), lambda i: (i, 0)),
    out_shape=jax.ShapeDtypeStruct((_N, 64), jnp.float32),
)


def _tcC_body(a0_ref, a1_ref, m2p_ref, dinv_ref, b2_ref,
              fw1_ref, fb1_ref, fw2_ref, fb2_ref,
              asn_ref, pen_ref, s_ref):
    i = pl.program_id(0)
    h2 = dinv_ref[...] * (a0_ref[...] + a1_ref[...] + m2p_ref[...]) + b2_ref[...]
    t = jnp.tanh(jnp.dot(h2, fw1_ref[...], preferred_element_type=jnp.float32)
                 + fb1_ref[...])
    logits = jnp.dot(t, fw2_ref[...], preferred_element_type=jnp.float32) + fb2_ref[...]
    mx = jnp.max(logits, axis=1, keepdims=True)
    e = jnp.exp(logits - mx)
    asn = e / jnp.sum(e, axis=1, keepdims=True)
    asn_ref[...] = asn
    d = asn - 0.5
    s1 = jnp.sum(d)
    s2 = jnp.sum(d * d)

    @pl.when(i == 0)
    def _():
        s_ref[0] = s1
        s_ref[1] = s2

    @pl.when(i > 0)
    def _():
        s_ref[0] += s1
        s_ref[1] += s2

    @pl.when(i == pl.num_programs(0) - 1)
    def _():
        n = 2.0 * _N
        var = (s_ref[1] - s_ref[0] * s_ref[0] / n) / (n - 1.0)
        pen_ref[...] = jnp.full((1, 1), var, dtype=jnp.float32)


_tcC = pl.pallas_call(
    _tcC_body,
    grid=(_G,),
    in_specs=[
        pl.BlockSpec((_R, 64), lambda i: (i, 0)),
        pl.BlockSpec((_R, 64), lambda i: (i, 0)),
        pl.BlockSpec((_R, 64), lambda i: (i, 0)),
        pl.BlockSpec((_R, 1), lambda i: (i, 0)),
        pl.BlockSpec((1, 64), lambda i: (0, 0)),
        pl.BlockSpec((64, 32), lambda i: (0, 0)),
        pl.BlockSpec((1, 32), lambda i: (0, 0)),
        pl.BlockSpec((32, 2), lambda i: (0, 0)),
        pl.BlockSpec((1, 2), lambda i: (0, 0)),
    ],
    out_specs=[
        pl.BlockSpec((_R, 2), lambda i: (i, 0)),
        pl.BlockSpec((1, 1), lambda i: (0, 0)),
    ],
    out_shape=[
        jax.ShapeDtypeStruct((_N, 2), jnp.float32),
        jax.ShapeDtypeStruct((1, 1), jnp.float32),
    ],
    scratch_shapes=[pltpu.SMEM((2,), jnp.float32)],
)


def kernel(x, edge_index, W1, b1, W2, b2, fc1_W, fc1_b, fc2_W, fc2_b):
    src2 = edge_index[0].reshape(_E // _K, _K)
    dst2 = edge_index[1].reshape(_E // _K, _K)
    ones_k = jnp.ones((_K,), jnp.float32)
    z1d = jnp.zeros((1024,), jnp.float32)
    z128 = jnp.zeros((_K, 128), jnp.float32)
    z64 = jnp.zeros((_K, 64), jnp.float32)

    degp = _deg_call(dst2, ones_k, z1d)
    deg0 = degp[:_N].reshape(_N, 1)
    deg1 = degp[_N:].reshape(_N, 1)
    m1p, dinv = _tcA(deg0, deg1, x, W1)
    acc1 = _agg128(m1p, src2, dst2, z128)
    m2p = _tcB(acc1[0], acc1[1], m1p, dinv, b1.reshape(1, -1), W2)
    acc2 = _agg64(m2p, src2, dst2, z64)
    asn, pen = _tcC(acc2[0], acc2[1], m2p, dinv, b2.reshape(1, -1),
                    fc1_W, fc1_b.reshape(1, -1), fc2_W, fc2_b.reshape(1, -1))
    return asn, pen.reshape(())


# speedup vs baseline: 31.0352x; 1.0200x over previous
import jax
import jax.numpy as jnp
from jax import lax
from jax.experimental import pallas as pl
from jax.experimental.pallas import tpu as pltpu
from jax.experimental.pallas import tpu_sc as plsc

_N = 10000
_E = 320000
_NC = 2
_NS = 16
_K = 80
_EPT = _E // (_NC * _NS)
_NBLK = _EPT // _K
_RCH = _N // _K
_ZJ = (_RCH + _NS - 1) // _NS


def _sc_mesh():
    return plsc.VectorSubcoreMesh(core_axis_name="c", subcore_axis_name="s")


def _deg_body(dst2_hbm, ones_hbm, z1d_hbm, out_hbm, didx_v, ones_v, stg_v,
              deg_sh, sem_a, sem_b):
    c = lax.axis_index("c")
    s = lax.axis_index("s")
    wid = c * _NS + s
    cp = pltpu.async_copy(dst2_hbm.at[pl.ds(wid * _NBLK, _NBLK)], didx_v,
                          sem_a)
    pltpu.sync_copy(ones_hbm, ones_v)
    pltpu.sync_copy(z1d_hbm.at[pl.ds(0, 640)], stg_v)
    pltpu.sync_copy(stg_v.at[pl.ds(0, 624)], deg_sh.at[pl.ds(s * 624, 624)])

    @pl.when(s == 0)
    def _():
        pltpu.sync_copy(stg_v.at[pl.ds(0, 16)], deg_sh.at[pl.ds(9984, 16)])

    cp.wait()
    plsc.subcore_barrier()

    def _fire(i, sem):
        pltpu.async_copy(ones_v, deg_sh.at[didx_v.at[i]], sem, add=True)

    def _drain(i, sem):
        pltpu.make_async_copy(ones_v, deg_sh.at[didx_v.at[i]], sem).wait()

    _fire(0, sem_a)

    def body(j, carry):
        i0 = 2 * j
        i1 = 2 * j + 1
        i2 = 2 * j + 2

        @pl.when(i1 < _NBLK)
        def _():
            _fire(i1, sem_b)

        _drain(i0, sem_a)

        @pl.when(i2 < _NBLK)
        def _():
            _fire(i2, sem_a)

        @pl.when(i1 < _NBLK)
        def _():
            _drain(i1, sem_b)

        return carry

    lax.fori_loop(0, (_NBLK + 1) // 2, body, 0)
    plsc.subcore_barrier()
    pltpu.sync_copy(deg_sh.at[pl.ds(s * 624, 624)], stg_v.at[pl.ds(0, 624)])
    pltpu.sync_copy(stg_v.at[pl.ds(0, 624)],
                    out_hbm.at[pl.ds(c * _N + s * 624, 624)])

    @pl.when(s == 0)
    def _():
        pltpu.sync_copy(deg_sh.at[pl.ds(9984, 16)], stg_v.at[pl.ds(624, 16)])
        pltpu.sync_copy(stg_v.at[pl.ds(624, 16)],
                        out_hbm.at[pl.ds(c * _N + 9984, 16)])


_deg_call = pl.kernel(
    _deg_body,
    out_type=jax.ShapeDtypeStruct((_NC * _N,), jnp.float32),
    mesh=_sc_mesh(),
    scratch_types=[
        pltpu.VMEM((_NBLK, _K), jnp.int32),
        pltpu.VMEM((_K,), jnp.float32),
        pltpu.VMEM((640,), jnp.float32),
        pltpu.VMEM_SHARED((_N,), jnp.float32),
        pltpu.SemaphoreType.DMA,
        pltpu.SemaphoreType.DMA,
    ],
    compiler_params=pltpu.CompilerParams(use_tc_tiling_on_sc=False),
)


_NBUF = 3
_JMAIN = _NBLK // _NBUF


def _agg_body(m_hbm, src2_hbm, dst2_hbm, zrows_hbm, out_hbm,
              sidx_v, didx_v, r0, r1, r2, acc_sh,
              g0, g1, g2, s0, s1, s2):
    rows = (r0, r1, r2)
    gsem = (g0, g1, g2)
    ssem = (s0, s1, s2)
    c = lax.axis_index("c")
    s = lax.axis_index("s")
    wid = c * _NS + s

    cps = pltpu.async_copy(src2_hbm.at[pl.ds(wid * _NBLK, _NBLK)], sidx_v, g0)
    cpd = pltpu.async_copy(dst2_hbm.at[pl.ds(wid * _NBLK, _NBLK)], didx_v, g1)
    pltpu.sync_copy(zrows_hbm, r0)

    def zbody(j, carry):
        ch = s + j * _NS

        @pl.when(ch < _RCH)
        def _():
            pltpu.sync_copy(r0, acc_sh.at[pl.ds(ch * _K, _K)])

        return carry

    lax.fori_loop(0, _ZJ, zbody, 0)
    cps.wait()
    cpd.wait()
    plsc.subcore_barrier()

    def _gstart(i, t):
        pltpu.async_copy(m_hbm.at[sidx_v.at[i]], rows[t], gsem[t])

    def _gwait(i, t):
        pltpu.make_async_copy(m_hbm.at[sidx_v.at[i]], rows[t], gsem[t]).wait()

    def _sstart(i, t):
        pltpu.async_copy(rows[t], acc_sh.at[didx_v.at[i]], ssem[t], add=True)

    def _swait(i, t):
        pltpu.make_async_copy(rows[t], acc_sh.at[didx_v.at[i]],
                              ssem[t]).wait()

    for t in range(_NBUF):
        _gstart(t, t)

    def ebody(j, carry):
        base = _NBUF * j
        for t in range(_NBUF):
            i = base + t
            _gwait(i, t)
            _sstart(i, t)
        for t in range(_NBUF):
            i = base + t

            @pl.when(i + _NBUF < _NBLK)
            def _():
                _swait(i, t)
                _gstart(i + _NBUF, t)

        return carry

    lax.fori_loop(0, _JMAIN, ebody, 0)
    for i in range(_JMAIN * _NBUF, _NBLK):
        _gwait(i, i % _NBUF)
        _sstart(i, i % _NBUF)
    for i in range(_NBLK - _NBUF, _NBLK):
        _swait(i, i % _NBUF)
    plsc.subcore_barrier()

    def obody(j, carry):
        ch = s + j * _NS

        @pl.when(ch < _RCH)
        def _():
            pltpu.sync_copy(acc_sh.at[pl.ds(ch * _K, _K)], r0)
            pltpu.sync_copy(r0, out_hbm.at[c, pl.ds(ch * _K, _K)])

        return carry

    lax.fori_loop(0, _ZJ, obody, 0)


def _make_agg(d):
    return pl.kernel(
        _agg_body,
        out_type=jax.ShapeDtypeStruct((_NC, _N, d), jnp.float32),
        mesh=_sc_mesh(),
        scratch_types=[
            pltpu.VMEM((_NBLK, _K), jnp.int32),
            pltpu.VMEM((_NBLK, _K), jnp.int32),
            pltpu.VMEM((_K, d), jnp.float32),
            pltpu.VMEM((_K, d), jnp.float32),
            pltpu.VMEM((_K, d), jnp.float32),
            pltpu.VMEM_SHARED((_N, d), jnp.float32),
            pltpu.SemaphoreType.DMA,
            pltpu.SemaphoreType.DMA,
            pltpu.SemaphoreType.DMA,
            pltpu.SemaphoreType.DMA,
            pltpu.SemaphoreType.DMA,
            pltpu.SemaphoreType.DMA,
        ],
        compiler_params=pltpu.CompilerParams(use_tc_tiling_on_sc=False),
    )


_agg128 = _make_agg(128)
_agg64 = _make_agg(64)


_R = 1000
_G = _N // _R


def _tcA_body(deg0_ref, deg1_ref, x_ref, w1_ref, m1p_ref, dinv_ref):
    deg = deg0_ref[...] + deg1_ref[...] + 1.0
    dinv = 1.0 / jnp.sqrt(deg)
    m1 = jnp.dot(x_ref[...], w1_ref[...], preferred_element_type=jnp.float32)
    m1p_ref[...] = m1 * dinv
    dinv_ref[...] = dinv


_tcA = pl.pallas_call(
    _tcA_body,
    grid=(_G,),
    in_specs=[
        pl.BlockSpec((_R, 1), lambda i: (i, 0)),
        pl.BlockSpec((_R, 1), lambda i: (i, 0)),
        pl.BlockSpec((_R, 128), lambda i: (i, 0)),
        pl.BlockSpec((128, 128), lambda i: (0, 0)),
    ],
    out_specs=[
        pl.BlockSpec((_R, 128), lambda i: (i, 0)),
        pl.BlockSpec((_R, 1), lambda i: (i, 0)),
    ],
    out_shape=[
        jax.ShapeDtypeStruct((_N, 128), jnp.float32),
        jax.ShapeDtypeStruct((_N, 1), jnp.float32),
    ],
)


def _tcB_body(a0_ref, a1_ref, m1p_ref, dinv_ref, b1_ref, w2_ref, m2p_ref):
    dinv = dinv_ref[...]
    pre = dinv * (a0_ref[...] + a1_ref[...] + m1p_ref[...]) + b1_ref[...]
    h1 = jnp.maximum(pre, 0.0)
    m2 = jnp.dot(h1, w2_ref[...], preferred_element_type=jnp.float32)
    m2p_ref[...] = m2 * dinv


_tcB = pl.pallas_call(
    _tcB_body,
    grid=(_G,),
    in_specs=[
        pl.BlockSpec((_R, 128), lambda i: (i, 0)),
        pl.BlockSpec((_R, 128), lambda i: (i, 0)),
        pl.BlockSpec((_R, 128), lambda i: (i, 0)),
        pl.BlockSpec((_R, 1), lambda i: (i, 0)),
        pl.BlockSpec((1, 128), lambda i: (0, 0)),
        pl.BlockSpec((128, 64), lambda i: (0, 0)),
    ],
    out_specs=pl.BlockSpec((_R, 64), lambda i: (i, 0)),
    out_shape=jax.ShapeDtypeStruct((_N, 64), jnp.float32),
)


def _tcC_body(a0_ref, a1_ref, m2p_ref, dinv_ref, b2_ref,
              fw1_ref, fb1_ref, fw2_ref, fb2_ref,
              asn_ref, pen_ref, s_ref):
    i = pl.program_id(0)
    h2 = dinv_ref[...] * (a0_ref[...] + a1_ref[...] + m2p_ref[...]) + b2_ref[...]
    t = jnp.tanh(jnp.dot(h2, fw1_ref[...], preferred_element_type=jnp.float32)
                 + fb1_ref[...])
    logits = jnp.dot(t, fw2_ref[...], preferred_element_type=jnp.float32) + fb2_ref[...]
    mx = jnp.max(logits, axis=1, keepdims=True)
    e = jnp.exp(logits - mx)
    asn = e / jnp.sum(e, axis=1, keepdims=True)
    asn_ref[...] = asn
    d = asn - 0.5
    s1 = jnp.sum(d)
    s2 = jnp.sum(d * d)

    @pl.when(i == 0)
    def _():
        s_ref[0] = s1
        s_ref[1] = s2

    @pl.when(i > 0)
    def _():
        s_ref[0] += s1
        s_ref[1] += s2

    @pl.when(i == pl.num_programs(0) - 1)
    def _():
        n = 2.0 * _N
        var = (s_ref[1] - s_ref[0] * s_ref[0] / n) / (n - 1.0)
        pen_ref[...] = jnp.full((1, 1), var, dtype=jnp.float32)


_tcC = pl.pallas_call(
    _tcC_body,
    grid=(_G,),
    in_specs=[
        pl.BlockSpec((_R, 64), lambda i: (i, 0)),
        pl.BlockSpec((_R, 64), lambda i: (i, 0)),
        pl.BlockSpec((_R, 64), lambda i: (i, 0)),
        pl.BlockSpec((_R, 1), lambda i: (i, 0)),
        pl.BlockSpec((1, 64), lambda i: (0, 0)),
        pl.BlockSpec((64, 32), lambda i: (0, 0)),
        pl.BlockSpec((1, 32), lambda i: (0, 0)),
        pl.BlockSpec((32, 2), lambda i: (0, 0)),
        pl.BlockSpec((1, 2), lambda i: (0, 0)),
    ],
    out_specs=[
        pl.BlockSpec((_R, 2), lambda i: (i, 0)),
        pl.BlockSpec((1, 1), lambda i: (0, 0)),
    ],
    out_shape=[
        jax.ShapeDtypeStruct((_N, 2), jnp.float32),
        jax.ShapeDtypeStruct((1, 1), jnp.float32),
    ],
    scratch_shapes=[pltpu.SMEM((2,), jnp.float32)],
)


def kernel(x, edge_index, W1, b1, W2, b2, fc1_W, fc1_b, fc2_W, fc2_b):
    src2 = edge_index[0].reshape(_E // _K, _K)
    dst2 = edge_index[1].reshape(_E // _K, _K)
    ones_k = jnp.ones((_K,), jnp.float32)
    z1d = jnp.zeros((1024,), jnp.float32)
    z128 = jnp.zeros((_K, 128), jnp.float32)
    z64 = jnp.zeros((_K, 64), jnp.float32)

    degp = _deg_call(dst2, ones_k, z1d)
    deg0 = degp[:_N].reshape(_N, 1)
    deg1 = degp[_N:].reshape(_N, 1)
    m1p, dinv = _tcA(deg0, deg1, x, W1)
    acc1 = _agg128(m1p, src2, dst2, z128)
    m2p = _tcB(acc1[0], acc1[1], m1p, dinv, b1.reshape(1, -1), W2)
    acc2 = _agg64(m2p, src2, dst2, z64)
    asn, pen = _tcC(acc2[0], acc2[1], m2p, dinv, b2.reshape(1, -1),
                    fc1_W, fc1_b.reshape(1, -1), fc2_W, fc2_b.reshape(1, -1))
    return asn, pen.reshape(())


# speedup vs baseline: 35.6722x; 1.1494x over previous
import jax
import jax.numpy as jnp
from jax import lax
from jax.experimental import pallas as pl
from jax.experimental.pallas import tpu as pltpu
from jax.experimental.pallas import tpu_sc as plsc

_N = 10000
_E = 320000
_NC = 2
_NS = 16
_K = 80
_EPT = _E // (_NC * _NS)
_NBLK = _EPT // _K
_RCH = _N // _K
_ZJ = (_RCH + _NS - 1) // _NS


def _sc_mesh():
    return plsc.VectorSubcoreMesh(core_axis_name="c", subcore_axis_name="s")


def _deg_body(dst2_hbm, ones_hbm, z1d_hbm, out_hbm, didx_v, ones_v, stg_v,
              deg_sh, sem_a, sem_b):
    c = lax.axis_index("c")
    s = lax.axis_index("s")
    wid = c * _NS + s
    cp = pltpu.async_copy(dst2_hbm.at[pl.ds(wid * _NBLK, _NBLK)], didx_v,
                          sem_a)
    pltpu.sync_copy(ones_hbm, ones_v)
    pltpu.sync_copy(z1d_hbm.at[pl.ds(0, 640)], stg_v)
    pltpu.sync_copy(stg_v.at[pl.ds(0, 624)], deg_sh.at[pl.ds(s * 624, 624)])

    @pl.when(s == 0)
    def _():
        pltpu.sync_copy(stg_v.at[pl.ds(0, 16)], deg_sh.at[pl.ds(9984, 16)])

    cp.wait()
    plsc.subcore_barrier()

    def _fire(i, sem):
        pltpu.async_copy(ones_v, deg_sh.at[didx_v.at[i]], sem, add=True)

    def _drain(i, sem):
        pltpu.make_async_copy(ones_v, deg_sh.at[didx_v.at[i]], sem).wait()

    _fire(0, sem_a)

    def body(j, carry):
        i0 = 2 * j
        i1 = 2 * j + 1
        i2 = 2 * j + 2

        @pl.when(i1 < _NBLK)
        def _():
            _fire(i1, sem_b)

        _drain(i0, sem_a)

        @pl.when(i2 < _NBLK)
        def _():
            _fire(i2, sem_a)

        @pl.when(i1 < _NBLK)
        def _():
            _drain(i1, sem_b)

        return carry

    lax.fori_loop(0, (_NBLK + 1) // 2, body, 0)
    plsc.subcore_barrier()
    pltpu.sync_copy(deg_sh.at[pl.ds(s * 624, 624)], stg_v.at[pl.ds(0, 624)])
    pltpu.sync_copy(stg_v.at[pl.ds(0, 624)],
                    out_hbm.at[pl.ds(c * _N + s * 624, 624)])

    @pl.when(s == 0)
    def _():
        pltpu.sync_copy(deg_sh.at[pl.ds(9984, 16)], stg_v.at[pl.ds(624, 16)])
        pltpu.sync_copy(stg_v.at[pl.ds(624, 16)],
                        out_hbm.at[pl.ds(c * _N + 9984, 16)])


_deg_call = pl.kernel(
    _deg_body,
    out_type=jax.ShapeDtypeStruct((_NC * _N,), jnp.float32),
    mesh=_sc_mesh(),
    scratch_types=[
        pltpu.VMEM((_NBLK, _K), jnp.int32),
        pltpu.VMEM((_K,), jnp.float32),
        pltpu.VMEM((640,), jnp.float32),
        pltpu.VMEM_SHARED((_N,), jnp.float32),
        pltpu.SemaphoreType.DMA,
        pltpu.SemaphoreType.DMA,
    ],
    compiler_params=pltpu.CompilerParams(use_tc_tiling_on_sc=False),
)


_NBUF = 3
_JMAIN = _NBLK // _NBUF


def _agg_body(m_hbm, src2_hbm, dst2_hbm, zrows_hbm, out_hbm,
              sidx_v, didx_v, r0, r1, r2, acc_sh,
              g0, g1, g2, s0, s1, s2):
    rows = (r0, r1, r2)
    gsem = (g0, g1, g2)
    ssem = (s0, s1, s2)
    c = lax.axis_index("c")
    s = lax.axis_index("s")
    wid = c * _NS + s

    cps = pltpu.async_copy(src2_hbm.at[pl.ds(wid * _NBLK, _NBLK)], sidx_v, g0)
    cpd = pltpu.async_copy(dst2_hbm.at[pl.ds(wid * _NBLK, _NBLK)], didx_v, g1)
    pltpu.sync_copy(zrows_hbm, r0)

    def zbody(j, carry):
        ch = s + j * _NS

        @pl.when(ch < _RCH)
        def _():
            pltpu.sync_copy(r0, acc_sh.at[pl.ds(ch * _K, _K)])

        return carry

    lax.fori_loop(0, _ZJ, zbody, 0)
    cps.wait()
    cpd.wait()
    plsc.subcore_barrier()

    def _gstart(i, t):
        pltpu.async_copy(m_hbm.at[sidx_v.at[i]], rows[t], gsem[t])

    def _gwait(i, t):
        pltpu.make_async_copy(m_hbm.at[sidx_v.at[i]], rows[t], gsem[t]).wait()

    def _sstart(i, t):
        pltpu.async_copy(rows[t], acc_sh.at[didx_v.at[i]], ssem[t], add=True)

    def _swait(i, t):
        pltpu.make_async_copy(rows[t], acc_sh.at[didx_v.at[i]],
                              ssem[t]).wait()

    for t in range(_NBUF):
        _gstart(t, t)

    def ebody(j, carry):
        base = _NBUF * j
        for t in range(_NBUF):
            i = base + t
            _gwait(i, t)
            _sstart(i, t)
        for t in range(_NBUF):
            i = base + t

            @pl.when(i + _NBUF < _NBLK)
            def _():
                _swait(i, t)
                _gstart(i + _NBUF, t)

        return carry

    lax.fori_loop(0, _JMAIN, ebody, 0)
    for i in range(_JMAIN * _NBUF, _NBLK):
        _gwait(i, i % _NBUF)
        _sstart(i, i % _NBUF)
    for i in range(_NBLK - _NBUF, _NBLK):
        _swait(i, i % _NBUF)
    plsc.subcore_barrier()

    def obody(j, carry):
        ch = s + j * _NS

        @pl.when(ch < _RCH)
        def _():
            pltpu.sync_copy(acc_sh.at[pl.ds(ch * _K, _K)], r0)
            pltpu.sync_copy(r0, out_hbm.at[c, pl.ds(ch * _K, _K)])

        return carry

    lax.fori_loop(0, _ZJ, obody, 0)


def _make_agg(d):
    return pl.kernel(
        _agg_body,
        out_type=jax.ShapeDtypeStruct((_NC, _N, d), jnp.bfloat16),
        mesh=_sc_mesh(),
        scratch_types=[
            pltpu.VMEM((_NBLK, _K), jnp.int32),
            pltpu.VMEM((_NBLK, _K), jnp.int32),
            pltpu.VMEM((_K, d), jnp.bfloat16),
            pltpu.VMEM((_K, d), jnp.bfloat16),
            pltpu.VMEM((_K, d), jnp.bfloat16),
            pltpu.VMEM_SHARED((_N, d), jnp.bfloat16),
            pltpu.SemaphoreType.DMA,
            pltpu.SemaphoreType.DMA,
            pltpu.SemaphoreType.DMA,
            pltpu.SemaphoreType.DMA,
            pltpu.SemaphoreType.DMA,
            pltpu.SemaphoreType.DMA,
        ],
        compiler_params=pltpu.CompilerParams(use_tc_tiling_on_sc=False),
    )


_agg128 = _make_agg(128)
_agg64 = _make_agg(64)


_R = 1000
_G = _N // _R


def _tcA_body(deg0_ref, deg1_ref, x_ref, w1_ref, m1p_ref, m1pb_ref, dinv_ref):
    deg = deg0_ref[...] + deg1_ref[...] + 1.0
    dinv = 1.0 / jnp.sqrt(deg)
    m1 = jnp.dot(x_ref[...], w1_ref[...], preferred_element_type=jnp.float32)
    m1p = m1 * dinv
    m1p_ref[...] = m1p
    m1pb_ref[...] = m1p.astype(jnp.bfloat16)
    dinv_ref[...] = dinv


_tcA = pl.pallas_call(
    _tcA_body,
    grid=(_G,),
    in_specs=[
        pl.BlockSpec((_R, 1), lambda i: (i, 0)),
        pl.BlockSpec((_R, 1), lambda i: (i, 0)),
        pl.BlockSpec((_R, 128), lambda i: (i, 0)),
        pl.BlockSpec((128, 128), lambda i: (0, 0)),
    ],
    out_specs=[
        pl.BlockSpec((_R, 128), lambda i: (i, 0)),
        pl.BlockSpec((_R, 128), lambda i: (i, 0)),
        pl.BlockSpec((_R, 1), lambda i: (i, 0)),
    ],
    out_shape=[
        jax.ShapeDtypeStruct((_N, 128), jnp.float32),
        jax.ShapeDtypeStruct((_N, 128), jnp.bfloat16),
        jax.ShapeDtypeStruct((_N, 1), jnp.float32),
    ],
)


def _tcB_body(a0_ref, a1_ref, m1p_ref, dinv_ref, b1_ref, w2_ref,
              m2p_ref, m2pb_ref):
    dinv = dinv_ref[...]
    agg = a0_ref[...].astype(jnp.float32) + a1_ref[...].astype(jnp.float32)
    pre = dinv * (agg + m1p_ref[...]) + b1_ref[...]
    h1 = jnp.maximum(pre, 0.0)
    m2 = jnp.dot(h1, w2_ref[...], preferred_element_type=jnp.float32)
    m2p = m2 * dinv
    m2p_ref[...] = m2p
    m2pb_ref[...] = m2p.astype(jnp.bfloat16)


_tcB = pl.pallas_call(
    _tcB_body,
    grid=(_G,),
    in_specs=[
        pl.BlockSpec((_R, 128), lambda i: (i, 0)),
        pl.BlockSpec((_R, 128), lambda i: (i, 0)),
        pl.BlockSpec((_R, 128), lambda i: (i, 0)),
        pl.BlockSpec((_R, 1), lambda i: (i, 0)),
        pl.BlockSpec((1, 128), lambda i: (0, 0)),
        pl.BlockSpec((128, 64), lambda i: (0, 0)),
    ],
    out_specs=[
        pl.BlockSpec((_R, 64), lambda i: (i, 0)),
        pl.BlockSpec((_R, 64), lambda i: (i, 0)),
    ],
    out_shape=[
        jax.ShapeDtypeStruct((_N, 64), jnp.float32),
        jax.ShapeDtypeStruct((_N, 64), jnp.bfloat16),
    ],
)


def _tcC_body(a0_ref, a1_ref, m2p_ref, dinv_ref, b2_ref,
              fw1_ref, fb1_ref, fw2_ref, fb2_ref,
              asn_ref, pen_ref, s_ref):
    i = pl.program_id(0)
    agg = a0_ref[...].astype(jnp.float32) + a1_ref[...].astype(jnp.float32)
    h2 = dinv_ref[...] * (agg + m2p_ref[...]) + b2_ref[...]
    t = jnp.tanh(jnp.dot(h2, fw1_ref[...], preferred_element_type=jnp.float32)
                 + fb1_ref[...])
    logits = jnp.dot(t, fw2_ref[...], preferred_element_type=jnp.float32) + fb2_ref[...]
    mx = jnp.max(logits, axis=1, keepdims=True)
    e = jnp.exp(logits - mx)
    asn = e / jnp.sum(e, axis=1, keepdims=True)
    asn_ref[...] = asn
    d = asn - 0.5
    s1 = jnp.sum(d)
    s2 = jnp.sum(d * d)

    @pl.when(i == 0)
    def _():
        s_ref[0] = s1
        s_ref[1] = s2

    @pl.when(i > 0)
    def _():
        s_ref[0] += s1
        s_ref[1] += s2

    @pl.when(i == pl.num_programs(0) - 1)
    def _():
        n = 2.0 * _N
        var = (s_ref[1] - s_ref[0] * s_ref[0] / n) / (n - 1.0)
        pen_ref[...] = jnp.full((1, 1), var, dtype=jnp.float32)


_tcC = pl.pallas_call(
    _tcC_body,
    grid=(_G,),
    in_specs=[
        pl.BlockSpec((_R, 64), lambda i: (i, 0)),
        pl.BlockSpec((_R, 64), lambda i: (i, 0)),
        pl.BlockSpec((_R, 64), lambda i: (i, 0)),
        pl.BlockSpec((_R, 1), lambda i: (i, 0)),
        pl.BlockSpec((1, 64), lambda i: (0, 0)),
        pl.BlockSpec((64, 32), lambda i: (0, 0)),
        pl.BlockSpec((1, 32), lambda i: (0, 0)),
        pl.BlockSpec((32, 2), lambda i: (0, 0)),
        pl.BlockSpec((1, 2), lambda i: (0, 0)),
    ],
    out_specs=[
        pl.BlockSpec((_R, 2), lambda i: (i, 0)),
        pl.BlockSpec((1, 1), lambda i: (0, 0)),
    ],
    out_shape=[
        jax.ShapeDtypeStruct((_N, 2), jnp.float32),
        jax.ShapeDtypeStruct((1, 1), jnp.float32),
    ],
    scratch_shapes=[pltpu.SMEM((2,), jnp.float32)],
)


def kernel(x, edge_index, W1, b1, W2, b2, fc1_W, fc1_b, fc2_W, fc2_b):
    src2 = edge_index[0].reshape(_E // _K, _K)
    dst2 = edge_index[1].reshape(_E // _K, _K)
    ones_k = jnp.ones((_K,), jnp.float32)
    z1d = jnp.zeros((1024,), jnp.float32)
    z128 = jnp.zeros((_K, 128), jnp.bfloat16)
    z64 = jnp.zeros((_K, 64), jnp.bfloat16)

    degp = _deg_call(dst2, ones_k, z1d)
    deg0 = degp[:_N].reshape(_N, 1)
    deg1 = degp[_N:].reshape(_N, 1)
    m1p, m1pb, dinv = _tcA(deg0, deg1, x, W1)
    acc1 = _agg128(m1pb, src2, dst2, z128)
    m2p, m2pb = _tcB(acc1[0], acc1[1], m1p, dinv, b1.reshape(1, -1), W2)
    acc2 = _agg64(m2pb, src2, dst2, z64)
    asn, pen = _tcC(acc2[0], acc2[1], m2p, dinv, b2.reshape(1, -1),
                    fc1_W, fc1_b.reshape(1, -1), fc2_W, fc2_b.reshape(1, -1))
    return asn, pen.reshape(())


# speedup vs baseline: 37.7717x; 1.0589x over previous
import functools

import jax
import jax.numpy as jnp
from jax import lax
from jax.experimental import pallas as pl
from jax.experimental.pallas import tpu as pltpu
from jax.experimental.pallas import tpu_sc as plsc

_N = 10000
_E = 320000
_NC = 2
_NS = 16
_K = 80
_EPT = _E // (_NC * _NS)
_NBLK = _EPT // _K
_RCH = _N // _K
_ZJ = (_RCH + _NS - 1) // _NS


def _sc_mesh():
    return plsc.VectorSubcoreMesh(core_axis_name="c", subcore_axis_name="s")


def _deg_body(dst2_hbm, ones_hbm, z1d_hbm, out_hbm, didx_v, ones_v, stg_v,
              deg_sh, sem_a, sem_b):
    c = lax.axis_index("c")
    s = lax.axis_index("s")
    wid = c * _NS + s
    cp = pltpu.async_copy(dst2_hbm.at[pl.ds(wid * _NBLK, _NBLK)], didx_v,
                          sem_a)
    pltpu.sync_copy(ones_hbm, ones_v)
    pltpu.sync_copy(z1d_hbm.at[pl.ds(0, 640)], stg_v)
    pltpu.sync_copy(stg_v.at[pl.ds(0, 624)], deg_sh.at[pl.ds(s * 624, 624)])

    @pl.when(s == 0)
    def _():
        pltpu.sync_copy(stg_v.at[pl.ds(0, 16)], deg_sh.at[pl.ds(9984, 16)])

    cp.wait()
    plsc.subcore_barrier()

    def _fire(i, sem):
        pltpu.async_copy(ones_v, deg_sh.at[didx_v.at[i]], sem, add=True)

    def _drain(i, sem):
        pltpu.make_async_copy(ones_v, deg_sh.at[didx_v.at[i]], sem).wait()

    _fire(0, sem_a)

    def body(j, carry):
        i0 = 2 * j
        i1 = 2 * j + 1
        i2 = 2 * j + 2

        @pl.when(i1 < _NBLK)
        def _():
            _fire(i1, sem_b)

        _drain(i0, sem_a)

        @pl.when(i2 < _NBLK)
        def _():
            _fire(i2, sem_a)

        @pl.when(i1 < _NBLK)
        def _():
            _drain(i1, sem_b)

        return carry

    lax.fori_loop(0, (_NBLK + 1) // 2, body, 0)
    plsc.subcore_barrier()
    pltpu.sync_copy(deg_sh.at[pl.ds(s * 624, 624)], stg_v.at[pl.ds(0, 624)])
    pltpu.sync_copy(stg_v.at[pl.ds(0, 624)],
                    out_hbm.at[pl.ds(c * _N + s * 624, 624)])

    @pl.when(s == 0)
    def _():
        pltpu.sync_copy(deg_sh.at[pl.ds(9984, 16)], stg_v.at[pl.ds(624, 16)])
        pltpu.sync_copy(stg_v.at[pl.ds(624, 16)],
                        out_hbm.at[pl.ds(c * _N + 9984, 16)])


_deg_call = pl.kernel(
    _deg_body,
    out_type=jax.ShapeDtypeStruct((_NC * _N,), jnp.float32),
    mesh=_sc_mesh(),
    scratch_types=[
        pltpu.VMEM((_NBLK, _K), jnp.int32),
        pltpu.VMEM((_K,), jnp.float32),
        pltpu.VMEM((640,), jnp.float32),
        pltpu.VMEM_SHARED((_N,), jnp.float32),
        pltpu.SemaphoreType.DMA,
        pltpu.SemaphoreType.DMA,
    ],
    compiler_params=pltpu.CompilerParams(use_tc_tiling_on_sc=False),
)


_NBUF = 4
_JMAIN = _NBLK // _NBUF


def _agg_body(m_hbm, src2_hbm, dst2_hbm, zrows_hbm, out_hbm,
              sidx_v, didx_v, r0, r1, r2, r3, acc_sh,
              g0, g1, g2, g3, s0, s1, s2, s3):
    rows = (r0, r1, r2, r3)
    gsem = (g0, g1, g2, g3)
    ssem = (s0, s1, s2, s3)
    c = lax.axis_index("c")
    s = lax.axis_index("s")
    wid = c * _NS + s

    cps = pltpu.async_copy(src2_hbm.at[pl.ds(wid * _NBLK, _NBLK)], sidx_v, g0)
    cpd = pltpu.async_copy(dst2_hbm.at[pl.ds(wid * _NBLK, _NBLK)], didx_v, g1)
    pltpu.sync_copy(zrows_hbm, r0)

    def zbody(j, carry):
        ch = s + j * _NS

        @pl.when(ch < _RCH)
        def _():
            pltpu.sync_copy(r0, acc_sh.at[pl.ds(ch * _K, _K)])

        return carry

    lax.fori_loop(0, _ZJ, zbody, 0)
    cps.wait()
    cpd.wait()
    plsc.subcore_barrier()

    def _gstart(i, t):
        pltpu.async_copy(m_hbm.at[sidx_v.at[i]], rows[t], gsem[t])

    def _gwait(i, t):
        pltpu.make_async_copy(m_hbm.at[sidx_v.at[i]], rows[t], gsem[t]).wait()

    def _sstart(i, t):
        pltpu.async_copy(rows[t], acc_sh.at[didx_v.at[i]], ssem[t], add=True)

    def _swait(i, t):
        pltpu.make_async_copy(rows[t], acc_sh.at[didx_v.at[i]],
                              ssem[t]).wait()

    for t in range(_NBUF):
        _gstart(t, t)

    def ebody(j, carry):
        base = _NBUF * j
        for t in range(_NBUF):
            i = base + t
            _gwait(i, t)
            _sstart(i, t)
        for t in range(_NBUF):
            i = base + t

            @pl.when(i + _NBUF < _NBLK)
            def _():
                _swait(i, t)
                _gstart(i + _NBUF, t)

        return carry

    lax.fori_loop(0, _JMAIN, ebody, 0)
    for i in range(_JMAIN * _NBUF, _NBLK):
        _gwait(i, i % _NBUF)
        _sstart(i, i % _NBUF)
    for i in range(_NBLK - _NBUF, _NBLK):
        _swait(i, i % _NBUF)
    plsc.subcore_barrier()

    def obody(j, carry):
        ch = s + j * _NS

        @pl.when(ch < _RCH)
        def _():
            pltpu.sync_copy(acc_sh.at[pl.ds(ch * _K, _K)], r0)
            pltpu.sync_copy(r0, out_hbm.at[c, pl.ds(ch * _K, _K)])

        return carry

    lax.fori_loop(0, _ZJ, obody, 0)


def _make_agg(d):
    return pl.kernel(
        _agg_body,
        out_type=jax.ShapeDtypeStruct((_NC, _N, d), jnp.bfloat16),
        mesh=_sc_mesh(),
        scratch_types=[
            pltpu.VMEM((_NBLK, _K), jnp.int32),
            pltpu.VMEM((_NBLK, _K), jnp.int32),
            pltpu.VMEM((_K, d), jnp.bfloat16),
            pltpu.VMEM((_K, d), jnp.bfloat16),
            pltpu.VMEM((_K, d), jnp.bfloat16),
            pltpu.VMEM((_K, d), jnp.bfloat16),
            pltpu.VMEM_SHARED((_N, d), jnp.bfloat16),
            pltpu.SemaphoreType.DMA,
            pltpu.SemaphoreType.DMA,
            pltpu.SemaphoreType.DMA,
            pltpu.SemaphoreType.DMA,
            pltpu.SemaphoreType.DMA,
            pltpu.SemaphoreType.DMA,
            pltpu.SemaphoreType.DMA,
            pltpu.SemaphoreType.DMA,
        ],
        compiler_params=pltpu.CompilerParams(use_tc_tiling_on_sc=False),
    )


_agg128 = _make_agg(128)
_agg64 = _make_agg(64)


_R = 1000
_G = _N // _R


def _tc_mm_body(x_ref, w1_ref, m1_ref):
    m1_ref[...] = jnp.dot(x_ref[...], w1_ref[...],
                          preferred_element_type=jnp.float32)


_tc_mm = pl.pallas_call(
    _tc_mm_body,
    grid=(_G,),
    in_specs=[
        pl.BlockSpec((_R, 128), lambda i: (i, 0)),
        pl.BlockSpec((128, 128), lambda i: (0, 0)),
    ],
    out_specs=pl.BlockSpec((_R, 128), lambda i: (i, 0)),
    out_shape=jax.ShapeDtypeStruct((_N, 128), jnp.float32),
)


def _tc_scale_body(deg0_ref, deg1_ref, m1_ref, m1p_ref, m1pb_ref, dinv_ref):
    deg = deg0_ref[...] + deg1_ref[...] + 1.0
    dinv = 1.0 / jnp.sqrt(deg)
    m1p = m1_ref[...] * dinv
    m1p_ref[...] = m1p
    m1pb_ref[...] = m1p.astype(jnp.bfloat16)
    dinv_ref[...] = dinv


_tc_scale = pl.pallas_call(
    _tc_scale_body,
    grid=(_G,),
    in_specs=[
        pl.BlockSpec((_R, 1), lambda i: (i, 0)),
        pl.BlockSpec((_R, 1), lambda i: (i, 0)),
        pl.BlockSpec((_R, 128), lambda i: (i, 0)),
    ],
    out_specs=[
        pl.BlockSpec((_R, 128), lambda i: (i, 0)),
        pl.BlockSpec((_R, 128), lambda i: (i, 0)),
        pl.BlockSpec((_R, 1), lambda i: (i, 0)),
    ],
    out_shape=[
        jax.ShapeDtypeStruct((_N, 128), jnp.float32),
        jax.ShapeDtypeStruct((_N, 128), jnp.bfloat16),
        jax.ShapeDtypeStruct((_N, 1), jnp.float32),
    ],
)


def _tcB_body(a0_ref, a1_ref, m1p_ref, dinv_ref, b1_ref, w2_ref,
              m2p_ref, m2pb_ref):
    dinv = dinv_ref[...]
    agg = a0_ref[...].astype(jnp.float32) + a1_ref[...].astype(jnp.float32)
    pre = dinv * (agg + m1p_ref[...]) + b1_ref[...]
    h1 = jnp.maximum(pre, 0.0)
    m2 = jnp.dot(h1, w2_ref[...], preferred_element_type=jnp.float32)
    m2p = m2 * dinv
    m2p_ref[...] = m2p
    m2pb_ref[...] = m2p.astype(jnp.bfloat16)


_tcB = pl.pallas_call(
    _tcB_body,
    grid=(_G,),
    in_specs=[
        pl.BlockSpec((_R, 128), lambda i: (i, 0)),
        pl.BlockSpec((_R, 128), lambda i: (i, 0)),
        pl.BlockSpec((_R, 128), lambda i: (i, 0)),
        pl.BlockSpec((_R, 1), lambda i: (i, 0)),
        pl.BlockSpec((1, 128), lambda i: (0, 0)),
        pl.BlockSpec((128, 64), lambda i: (0, 0)),
    ],
    out_specs=[
        pl.BlockSpec((_R, 64), lambda i: (i, 0)),
        pl.BlockSpec((_R, 64), lambda i: (i, 0)),
    ],
    out_shape=[
        jax.ShapeDtypeStruct((_N, 64), jnp.float32),
        jax.ShapeDtypeStruct((_N, 64), jnp.bfloat16),
    ],
)


def _tcC_body(a0_ref, a1_ref, m2p_ref, dinv_ref, b2_ref,
              fw1_ref, fb1_ref, fw2_ref, fb2_ref,
              asn_ref, pen_ref, s_ref):
    i = pl.program_id(0)
    agg = a0_ref[...].astype(jnp.float32) + a1_ref[...].astype(jnp.float32)
    h2 = dinv_ref[...] * (agg + m2p_ref[...]) + b2_ref[...]
    t = jnp.tanh(jnp.dot(h2, fw1_ref[...], preferred_element_type=jnp.float32)
                 + fb1_ref[...])
    logits = jnp.dot(t, fw2_ref[...], preferred_element_type=jnp.float32) + fb2_ref[...]
    mx = jnp.max(logits, axis=1, keepdims=True)
    e = jnp.exp(logits - mx)
    asn = e / jnp.sum(e, axis=1, keepdims=True)
    asn_ref[...] = asn
    d = asn - 0.5
    s1 = jnp.sum(d)
    s2 = jnp.sum(d * d)

    @pl.when(i == 0)
    def _():
        s_ref[0] = s1
        s_ref[1] = s2

    @pl.when(i > 0)
    def _():
        s_ref[0] += s1
        s_ref[1] += s2

    @pl.when(i == pl.num_programs(0) - 1)
    def _():
        n = 2.0 * _N
        var = (s_ref[1] - s_ref[0] * s_ref[0] / n) / (n - 1.0)
        pen_ref[...] = jnp.full((1, 1), var, dtype=jnp.float32)


_tcC = pl.pallas_call(
    _tcC_body,
    grid=(_G,),
    in_specs=[
        pl.BlockSpec((_R, 64), lambda i: (i, 0)),
        pl.BlockSpec((_R, 64), lambda i: (i, 0)),
        pl.BlockSpec((_R, 64), lambda i: (i, 0)),
        pl.BlockSpec((_R, 1), lambda i: (i, 0)),
        pl.BlockSpec((1, 64), lambda i: (0, 0)),
        pl.BlockSpec((64, 32), lambda i: (0, 0)),
        pl.BlockSpec((1, 32), lambda i: (0, 0)),
        pl.BlockSpec((32, 2), lambda i: (0, 0)),
        pl.BlockSpec((1, 2), lambda i: (0, 0)),
    ],
    out_specs=[
        pl.BlockSpec((_R, 2), lambda i: (i, 0)),
        pl.BlockSpec((1, 1), lambda i: (0, 0)),
    ],
    out_shape=[
        jax.ShapeDtypeStruct((_N, 2), jnp.float32),
        jax.ShapeDtypeStruct((1, 1), jnp.float32),
    ],
    scratch_shapes=[pltpu.SMEM((2,), jnp.float32)],
)


def kernel(x, edge_index, W1, b1, W2, b2, fc1_W, fc1_b, fc2_W, fc2_b):
    src2 = edge_index[0].reshape(_E // _K, _K)
    dst2 = edge_index[1].reshape(_E // _K, _K)
    ones_k = jnp.ones((_K,), jnp.float32)
    z1d = jnp.zeros((1024,), jnp.float32)
    z128 = jnp.zeros((_K, 128), jnp.bfloat16)
    z64 = jnp.zeros((_K, 64), jnp.bfloat16)

    m1 = _tc_mm(x, W1)
    degp = _deg_call(dst2, ones_k, z1d)
    deg0 = degp[:_N].reshape(_N, 1)
    deg1 = degp[_N:].reshape(_N, 1)
    m1p, m1pb, dinv = _tc_scale(deg0, deg1, m1)
    acc1 = _agg128(m1pb, src2, dst2, z128)
    m2p, m2pb = _tcB(acc1[0], acc1[1], m1p, dinv, b1.reshape(1, -1), W2)
    acc2 = _agg64(m2pb, src2, dst2, z64)
    asn, pen = _tcC(acc2[0], acc2[1], m2p, dinv, b2.reshape(1, -1),
                    fc1_W, fc1_b.reshape(1, -1), fc2_W, fc2_b.reshape(1, -1))
    return asn, pen.reshape(())


# speedup vs baseline: 39.3282x; 1.0412x over previous
import functools

import jax
import jax.numpy as jnp
from jax import lax
from jax.experimental import pallas as pl
from jax.experimental.pallas import tpu as pltpu
from jax.experimental.pallas import tpu_sc as plsc

_N = 10000
_E = 320000
_NC = 2
_NS = 16
_K = 80
_EPT = _E // (_NC * _NS)
_NBLK = _EPT // _K
_RCH = _N // _K
_ZJ = (_RCH + _NS - 1) // _NS


def _sc_mesh():
    return plsc.VectorSubcoreMesh(core_axis_name="c", subcore_axis_name="s")


def _deg_body(dst2_hbm, ones_hbm, z1d_hbm, out_hbm, didx_v, ones_v, stg_v,
              deg_sh, sem_a, sem_b):
    c = lax.axis_index("c")
    s = lax.axis_index("s")
    wid = c * _NS + s
    cp = pltpu.async_copy(dst2_hbm.at[pl.ds(wid * _NBLK, _NBLK)], didx_v,
                          sem_a)
    pltpu.sync_copy(ones_hbm, ones_v)
    pltpu.sync_copy(z1d_hbm.at[pl.ds(0, 640)], stg_v)
    pltpu.sync_copy(stg_v.at[pl.ds(0, 624)], deg_sh.at[pl.ds(s * 624, 624)])

    @pl.when(s == 0)
    def _():
        pltpu.sync_copy(stg_v.at[pl.ds(0, 16)], deg_sh.at[pl.ds(9984, 16)])

    cp.wait()
    plsc.subcore_barrier()

    def _fire(i, sem):
        pltpu.async_copy(ones_v, deg_sh.at[didx_v.at[i]], sem, add=True)

    def _drain(i, sem):
        pltpu.make_async_copy(ones_v, deg_sh.at[didx_v.at[i]], sem).wait()

    _fire(0, sem_a)

    def body(j, carry):
        i0 = 2 * j
        i1 = 2 * j + 1
        i2 = 2 * j + 2

        @pl.when(i1 < _NBLK)
        def _():
            _fire(i1, sem_b)

        _drain(i0, sem_a)

        @pl.when(i2 < _NBLK)
        def _():
            _fire(i2, sem_a)

        @pl.when(i1 < _NBLK)
        def _():
            _drain(i1, sem_b)

        return carry

    lax.fori_loop(0, (_NBLK + 1) // 2, body, 0)
    plsc.subcore_barrier()
    pltpu.sync_copy(deg_sh.at[pl.ds(s * 624, 624)], stg_v.at[pl.ds(0, 624)])
    pltpu.sync_copy(stg_v.at[pl.ds(0, 624)],
                    out_hbm.at[pl.ds(c * _N + s * 624, 624)])

    @pl.when(s == 0)
    def _():
        pltpu.sync_copy(deg_sh.at[pl.ds(9984, 16)], stg_v.at[pl.ds(624, 16)])
        pltpu.sync_copy(stg_v.at[pl.ds(624, 16)],
                        out_hbm.at[pl.ds(c * _N + 9984, 16)])


_deg_call = pl.kernel(
    _deg_body,
    out_type=jax.ShapeDtypeStruct((_NC * _N,), jnp.float32),
    mesh=_sc_mesh(),
    scratch_types=[
        pltpu.VMEM((_NBLK, _K), jnp.int32),
        pltpu.VMEM((_K,), jnp.float32),
        pltpu.VMEM((640,), jnp.float32),
        pltpu.VMEM_SHARED((_N,), jnp.float32),
        pltpu.SemaphoreType.DMA,
        pltpu.SemaphoreType.DMA,
    ],
    compiler_params=pltpu.CompilerParams(use_tc_tiling_on_sc=False),
)


_NBUF = 6
_JMAIN = _NBLK // _NBUF


def _agg_body(m_hbm, src2_hbm, dst2_hbm, zrows_hbm, out_hbm,
              sidx_v, didx_v, r0, r1, r2, r3, r4, r5, acc_sh,
              g0, g1, g2, g3, g4, g5, s0, s1, s2, s3, s4, s5):
    rows = (r0, r1, r2, r3, r4, r5)
    gsem = (g0, g1, g2, g3, g4, g5)
    ssem = (s0, s1, s2, s3, s4, s5)
    c = lax.axis_index("c")
    s = lax.axis_index("s")
    wid = c * _NS + s

    cps = pltpu.async_copy(src2_hbm.at[pl.ds(wid * _NBLK, _NBLK)], sidx_v, g0)
    cpd = pltpu.async_copy(dst2_hbm.at[pl.ds(wid * _NBLK, _NBLK)], didx_v, g1)
    pltpu.sync_copy(zrows_hbm, r0)

    def zbody(j, carry):
        ch = s + j * _NS

        @pl.when(ch < _RCH)
        def _():
            pltpu.sync_copy(r0, acc_sh.at[pl.ds(ch * _K, _K)])

        return carry

    lax.fori_loop(0, _ZJ, zbody, 0)
    cps.wait()
    cpd.wait()
    plsc.subcore_barrier()

    def _gstart(i, t):
        pltpu.async_copy(m_hbm.at[sidx_v.at[i]], rows[t], gsem[t])

    def _gwait(i, t):
        pltpu.make_async_copy(m_hbm.at[sidx_v.at[i]], rows[t], gsem[t]).wait()

    def _sstart(i, t):
        pltpu.async_copy(rows[t], acc_sh.at[didx_v.at[i]], ssem[t], add=True)

    def _swait(i, t):
        pltpu.make_async_copy(rows[t], acc_sh.at[didx_v.at[i]],
                              ssem[t]).wait()

    for t in range(_NBUF):
        _gstart(t, t)

    def ebody(j, carry):
        base = _NBUF * j
        for t in range(_NBUF):
            i = base + t
            _gwait(i, t)
            _sstart(i, t)
        for t in range(_NBUF):
            i = base + t

            @pl.when(i + _NBUF < _NBLK)
            def _():
                _swait(i, t)
                _gstart(i + _NBUF, t)

        return carry

    lax.fori_loop(0, _JMAIN, ebody, 0)
    for i in range(_JMAIN * _NBUF, _NBLK):
        _gwait(i, i % _NBUF)
        _sstart(i, i % _NBUF)
    for i in range(_NBLK - _NBUF, _NBLK):
        _swait(i, i % _NBUF)
    plsc.subcore_barrier()

    def obody(j, carry):
        ch = s + j * _NS

        @pl.when(ch < _RCH)
        def _():
            pltpu.sync_copy(acc_sh.at[pl.ds(ch * _K, _K)], r0)
            pltpu.sync_copy(r0, out_hbm.at[c, pl.ds(ch * _K, _K)])

        return carry

    lax.fori_loop(0, _ZJ, obody, 0)


def _make_agg(d):
    return pl.kernel(
        _agg_body,
        out_type=jax.ShapeDtypeStruct((_NC, _N, d), jnp.bfloat16),
        mesh=_sc_mesh(),
        scratch_types=[
            pltpu.VMEM((_NBLK, _K), jnp.int32),
            pltpu.VMEM((_NBLK, _K), jnp.int32),
            pltpu.VMEM((_K, d), jnp.bfloat16),
            pltpu.VMEM((_K, d), jnp.bfloat16),
            pltpu.VMEM((_K, d), jnp.bfloat16),
            pltpu.VMEM((_K, d), jnp.bfloat16),
            pltpu.VMEM((_K, d), jnp.bfloat16),
            pltpu.VMEM((_K, d), jnp.bfloat16),
            pltpu.VMEM_SHARED((_N, d), jnp.bfloat16),
            pltpu.SemaphoreType.DMA,
            pltpu.SemaphoreType.DMA,
            pltpu.SemaphoreType.DMA,
            pltpu.SemaphoreType.DMA,
            pltpu.SemaphoreType.DMA,
            pltpu.SemaphoreType.DMA,
            pltpu.SemaphoreType.DMA,
            pltpu.SemaphoreType.DMA,
            pltpu.SemaphoreType.DMA,
            pltpu.SemaphoreType.DMA,
            pltpu.SemaphoreType.DMA,
            pltpu.SemaphoreType.DMA,
        ],
        compiler_params=pltpu.CompilerParams(use_tc_tiling_on_sc=False),
    )


_agg128 = _make_agg(128)
_agg64 = _make_agg(64)


_R = 1000
_G = _N // _R


def _tc_mm_body(x_ref, w1_ref, m1_ref):
    m1_ref[...] = jnp.dot(x_ref[...], w1_ref[...],
                          preferred_element_type=jnp.float32)


_tc_mm = pl.pallas_call(
    _tc_mm_body,
    grid=(_G,),
    in_specs=[
        pl.BlockSpec((_R, 128), lambda i: (i, 0)),
        pl.BlockSpec((128, 128), lambda i: (0, 0)),
    ],
    out_specs=pl.BlockSpec((_R, 128), lambda i: (i, 0)),
    out_shape=jax.ShapeDtypeStruct((_N, 128), jnp.float32),
)


def _tc_scale_body(deg0_ref, deg1_ref, m1_ref, m1p_ref, m1pb_ref, dinv_ref):
    deg = deg0_ref[...] + deg1_ref[...] + 1.0
    dinv = 1.0 / jnp.sqrt(deg)
    m1p = m1_ref[...] * dinv
    m1p_ref[...] = m1p
    m1pb_ref[...] = m1p.astype(jnp.bfloat16)
    dinv_ref[...] = dinv


_tc_scale = pl.pallas_call(
    _tc_scale_body,
    grid=(_G,),
    in_specs=[
        pl.BlockSpec((_R, 1), lambda i: (i, 0)),
        pl.BlockSpec((_R, 1), lambda i: (i, 0)),
        pl.BlockSpec((_R, 128), lambda i: (i, 0)),
    ],
    out_specs=[
        pl.BlockSpec((_R, 128), lambda i: (i, 0)),
        pl.BlockSpec((_R, 128), lambda i: (i, 0)),
        pl.BlockSpec((_R, 1), lambda i: (i, 0)),
    ],
    out_shape=[
        jax.ShapeDtypeStruct((_N, 128), jnp.float32),
        jax.ShapeDtypeStruct((_N, 128), jnp.bfloat16),
        jax.ShapeDtypeStruct((_N, 1), jnp.float32),
    ],
)


def _tcB_body(a0_ref, a1_ref, m1p_ref, dinv_ref, b1_ref, w2_ref,
              m2p_ref, m2pb_ref):
    dinv = dinv_ref[...]
    agg = a0_ref[...].astype(jnp.float32) + a1_ref[...].astype(jnp.float32)
    pre = dinv * (agg + m1p_ref[...]) + b1_ref[...]
    h1 = jnp.maximum(pre, 0.0)
    m2 = jnp.dot(h1, w2_ref[...], preferred_element_type=jnp.float32)
    m2p = m2 * dinv
    m2p_ref[...] = m2p
    m2pb_ref[...] = m2p.astype(jnp.bfloat16)


_tcB = pl.pallas_call(
    _tcB_body,
    grid=(_G,),
    in_specs=[
        pl.BlockSpec((_R, 128), lambda i: (i, 0)),
        pl.BlockSpec((_R, 128), lambda i: (i, 0)),
        pl.BlockSpec((_R, 128), lambda i: (i, 0)),
        pl.BlockSpec((_R, 1), lambda i: (i, 0)),
        pl.BlockSpec((1, 128), lambda i: (0, 0)),
        pl.BlockSpec((128, 64), lambda i: (0, 0)),
    ],
    out_specs=[
        pl.BlockSpec((_R, 64), lambda i: (i, 0)),
        pl.BlockSpec((_R, 64), lambda i: (i, 0)),
    ],
    out_shape=[
        jax.ShapeDtypeStruct((_N, 64), jnp.float32),
        jax.ShapeDtypeStruct((_N, 64), jnp.bfloat16),
    ],
)


def _tcC_body(a0_ref, a1_ref, m2p_ref, dinv_ref, b2_ref,
              fw1_ref, fb1_ref, fw2_ref, fb2_ref,
              asn_ref, pen_ref, s_ref):
    i = pl.program_id(0)
    agg = a0_ref[...].astype(jnp.float32) + a1_ref[...].astype(jnp.float32)
    h2 = dinv_ref[...] * (agg + m2p_ref[...]) + b2_ref[...]
    t = jnp.tanh(jnp.dot(h2, fw1_ref[...], preferred_element_type=jnp.float32)
                 + fb1_ref[...])
    logits = jnp.dot(t, fw2_ref[...], preferred_element_type=jnp.float32) + fb2_ref[...]
    mx = jnp.max(logits, axis=1, keepdims=True)
    e = jnp.exp(logits - mx)
    asn = e / jnp.sum(e, axis=1, keepdims=True)
    asn_ref[...] = asn
    d = asn - 0.5
    s1 = jnp.sum(d)
    s2 = jnp.sum(d * d)

    @pl.when(i == 0)
    def _():
        s_ref[0] = s1
        s_ref[1] = s2

    @pl.when(i > 0)
    def _():
        s_ref[0] += s1
        s_ref[1] += s2

    @pl.when(i == pl.num_programs(0) - 1)
    def _():
        n = 2.0 * _N
        var = (s_ref[1] - s_ref[0] * s_ref[0] / n) / (n - 1.0)
        pen_ref[...] = jnp.full((1, 1), var, dtype=jnp.float32)


_tcC = pl.pallas_call(
    _tcC_body,
    grid=(_G,),
    in_specs=[
        pl.BlockSpec((_R, 64), lambda i: (i, 0)),
        pl.BlockSpec((_R, 64), lambda i: (i, 0)),
        pl.BlockSpec((_R, 64), lambda i: (i, 0)),
        pl.BlockSpec((_R, 1), lambda i: (i, 0)),
        pl.BlockSpec((1, 64), lambda i: (0, 0)),
        pl.BlockSpec((64, 32), lambda i: (0, 0)),
        pl.BlockSpec((1, 32), lambda i: (0, 0)),
        pl.BlockSpec((32, 2), lambda i: (0, 0)),
        pl.BlockSpec((1, 2), lambda i: (0, 0)),
    ],
    out_specs=[
        pl.BlockSpec((_R, 2), lambda i: (i, 0)),
        pl.BlockSpec((1, 1), lambda i: (0, 0)),
    ],
    out_shape=[
        jax.ShapeDtypeStruct((_N, 2), jnp.float32),
        jax.ShapeDtypeStruct((1, 1), jnp.float32),
    ],
    scratch_shapes=[pltpu.SMEM((2,), jnp.float32)],
)


def kernel(x, edge_index, W1, b1, W2, b2, fc1_W, fc1_b, fc2_W, fc2_b):
    src2 = edge_index[0].reshape(_E // _K, _K)
    dst2 = edge_index[1].reshape(_E // _K, _K)
    ones_k = jnp.ones((_K,), jnp.float32)
    z1d = jnp.zeros((1024,), jnp.float32)
    z128 = jnp.zeros((_K, 128), jnp.bfloat16)
    z64 = jnp.zeros((_K, 64), jnp.bfloat16)

    m1 = _tc_mm(x, W1)
    degp = _deg_call(dst2, ones_k, z1d)
    deg0 = degp[:_N].reshape(_N, 1)
    deg1 = degp[_N:].reshape(_N, 1)
    m1p, m1pb, dinv = _tc_scale(deg0, deg1, m1)
    acc1 = _agg128(m1pb, src2, dst2, z128)
    m2p, m2pb = _tcB(acc1[0], acc1[1], m1p, dinv, b1.reshape(1, -1), W2)
    acc2 = _agg64(m2pb, src2, dst2, z64)
    asn, pen = _tcC(acc2[0], acc2[1], m2p, dinv, b2.reshape(1, -1),
                    fc1_W, fc1_b.reshape(1, -1), fc2_W, fc2_b.reshape(1, -1))
    return asn, pen.reshape(())


# speedup vs baseline: 41.2943x; 1.0500x over previous
import functools

import jax
import jax.numpy as jnp
from jax import lax
from jax.experimental import pallas as pl
from jax.experimental.pallas import tpu as pltpu
from jax.experimental.pallas import tpu_sc as plsc

_N = 10000
_E = 320000
_NC = 2
_NS = 16
_K = 80
_EPT = _E // (_NC * _NS)
_NBLK = _EPT // _K
_RCH = _N // _K
_ZJ = (_RCH + _NS - 1) // _NS


def _sc_mesh():
    return plsc.VectorSubcoreMesh(core_axis_name="c", subcore_axis_name="s")


def _deg_body(ei3_hbm, ones_hbm, z1d_hbm, out_hbm, didx_v, ones_v, stg_v,
              deg_sh, sem_a, sem_b):
    c = lax.axis_index("c")
    s = lax.axis_index("s")
    wid = c * _NS + s
    cp = pltpu.async_copy(ei3_hbm.at[1, pl.ds(wid * _NBLK, _NBLK)], didx_v,
                          sem_a)
    pltpu.sync_copy(ones_hbm, ones_v)
    pltpu.sync_copy(z1d_hbm.at[pl.ds(0, 640)], stg_v)
    pltpu.sync_copy(stg_v.at[pl.ds(0, 624)], deg_sh.at[pl.ds(s * 624, 624)])

    @pl.when(s == 0)
    def _():
        pltpu.sync_copy(stg_v.at[pl.ds(0, 16)], deg_sh.at[pl.ds(9984, 16)])

    cp.wait()
    plsc.subcore_barrier()

    def _fire(i, sem):
        pltpu.async_copy(ones_v, deg_sh.at[didx_v.at[i]], sem, add=True)

    def _drain(i, sem):
        pltpu.make_async_copy(ones_v, deg_sh.at[didx_v.at[i]], sem).wait()

    _fire(0, sem_a)

    def body(j, carry):
        i0 = 2 * j
        i1 = 2 * j + 1
        i2 = 2 * j + 2

        @pl.when(i1 < _NBLK)
        def _():
            _fire(i1, sem_b)

        _drain(i0, sem_a)

        @pl.when(i2 < _NBLK)
        def _():
            _fire(i2, sem_a)

        @pl.when(i1 < _NBLK)
        def _():
            _drain(i1, sem_b)

        return carry

    lax.fori_loop(0, (_NBLK + 1) // 2, body, 0)
    plsc.subcore_barrier()
    pltpu.sync_copy(deg_sh.at[pl.ds(s * 624, 624)], stg_v.at[pl.ds(0, 624)])
    pltpu.sync_copy(stg_v.at[pl.ds(0, 624)],
                    out_hbm.at[pl.ds(c * _N + s * 624, 624)])

    @pl.when(s == 0)
    def _():
        pltpu.sync_copy(deg_sh.at[pl.ds(9984, 16)], stg_v.at[pl.ds(624, 16)])
        pltpu.sync_copy(stg_v.at[pl.ds(624, 16)],
                        out_hbm.at[pl.ds(c * _N + 9984, 16)])


_deg_call = pl.kernel(
    _deg_body,
    out_type=jax.ShapeDtypeStruct((_NC * _N,), jnp.float32),
    mesh=_sc_mesh(),
    scratch_types=[
        pltpu.VMEM((_NBLK, _K), jnp.int32),
        pltpu.VMEM((_K,), jnp.float32),
        pltpu.VMEM((640,), jnp.float32),
        pltpu.VMEM_SHARED((_N,), jnp.float32),
        pltpu.SemaphoreType.DMA,
        pltpu.SemaphoreType.DMA,
    ],
    compiler_params=pltpu.CompilerParams(use_tc_tiling_on_sc=False),
)


_NBUF = 6
_JMAIN = _NBLK // _NBUF


def _agg_body(m_hbm, ei3_hbm, zrows_hbm, out_hbm,
              sidx_v, didx_v, r0, r1, r2, r3, r4, r5, acc_sh,
              g0, g1, g2, g3, g4, g5, s0, s1, s2, s3, s4, s5):
    rows = (r0, r1, r2, r3, r4, r5)
    gsem = (g0, g1, g2, g3, g4, g5)
    ssem = (s0, s1, s2, s3, s4, s5)
    c = lax.axis_index("c")
    s = lax.axis_index("s")
    wid = c * _NS + s

    cps = pltpu.async_copy(ei3_hbm.at[0, pl.ds(wid * _NBLK, _NBLK)], sidx_v,
                           g0)
    cpd = pltpu.async_copy(ei3_hbm.at[1, pl.ds(wid * _NBLK, _NBLK)], didx_v,
                           g1)
    pltpu.sync_copy(zrows_hbm, r0)

    def zbody(j, carry):
        ch = s + j * _NS

        @pl.when(ch < _RCH)
        def _():
            pltpu.sync_copy(r0, acc_sh.at[pl.ds(ch * _K, _K)])

        return carry

    lax.fori_loop(0, _ZJ, zbody, 0)
    cps.wait()
    cpd.wait()
    plsc.subcore_barrier()

    def _gstart(i, t):
        pltpu.async_copy(m_hbm.at[sidx_v.at[i]], rows[t], gsem[t])

    def _gwait(i, t):
        pltpu.make_async_copy(m_hbm.at[sidx_v.at[i]], rows[t], gsem[t]).wait()

    def _sstart(i, t):
        pltpu.async_copy(rows[t], acc_sh.at[didx_v.at[i]], ssem[t], add=True)

    def _swait(i, t):
        pltpu.make_async_copy(rows[t], acc_sh.at[didx_v.at[i]],
                              ssem[t]).wait()

    for t in range(_NBUF):
        _gstart(t, t)

    def ebody(j, carry):
        base = _NBUF * j
        for t in range(_NBUF):
            i = base + t
            _gwait(i, t)
            _sstart(i, t)
        for t in range(_NBUF):
            i = base + t

            @pl.when(i + _NBUF < _NBLK)
            def _():
                _swait(i, t)
                _gstart(i + _NBUF, t)

        return carry

    lax.fori_loop(0, _JMAIN, ebody, 0)
    for i in range(_JMAIN * _NBUF, _NBLK):
        _gwait(i, i % _NBUF)
        _sstart(i, i % _NBUF)
    for i in range(_NBLK - _NBUF, _NBLK):
        _swait(i, i % _NBUF)
    plsc.subcore_barrier()

    def obody(j, carry):
        ch = s + j * _NS

        @pl.when(ch < _RCH)
        def _():
            pltpu.sync_copy(acc_sh.at[pl.ds(ch * _K, _K)], r0)
            pltpu.sync_copy(r0, out_hbm.at[c, pl.ds(ch * _K, _K)])

        return carry

    lax.fori_loop(0, _ZJ, obody, 0)


def _make_agg(d):
    return pl.kernel(
        _agg_body,
        out_type=jax.ShapeDtypeStruct((_NC, _N, d), jnp.bfloat16),
        mesh=_sc_mesh(),
        scratch_types=[
            pltpu.VMEM((_NBLK, _K), jnp.int32),
            pltpu.VMEM((_NBLK, _K), jnp.int32),
            pltpu.VMEM((_K, d), jnp.bfloat16),
            pltpu.VMEM((_K, d), jnp.bfloat16),
            pltpu.VMEM((_K, d), jnp.bfloat16),
            pltpu.VMEM((_K, d), jnp.bfloat16),
            pltpu.VMEM((_K, d), jnp.bfloat16),
            pltpu.VMEM((_K, d), jnp.bfloat16),
            pltpu.VMEM_SHARED((_N, d), jnp.bfloat16),
            pltpu.SemaphoreType.DMA,
            pltpu.SemaphoreType.DMA,
            pltpu.SemaphoreType.DMA,
            pltpu.SemaphoreType.DMA,
            pltpu.SemaphoreType.DMA,
            pltpu.SemaphoreType.DMA,
            pltpu.SemaphoreType.DMA,
            pltpu.SemaphoreType.DMA,
            pltpu.SemaphoreType.DMA,
            pltpu.SemaphoreType.DMA,
            pltpu.SemaphoreType.DMA,
            pltpu.SemaphoreType.DMA,
        ],
        compiler_params=pltpu.CompilerParams(use_tc_tiling_on_sc=False),
    )


_agg128 = _make_agg(128)
_agg64 = _make_agg(64)


_R = 1000
_G = _N // _R


def _tc_mm_body(x_ref, w1_ref, m1_ref):
    m1_ref[...] = jnp.dot(x_ref[...], w1_ref[...],
                          preferred_element_type=jnp.float32)


_tc_mm = pl.pallas_call(
    _tc_mm_body,
    grid=(_G,),
    in_specs=[
        pl.BlockSpec((_R, 128), lambda i: (i, 0)),
        pl.BlockSpec((128, 128), lambda i: (0, 0)),
    ],
    out_specs=pl.BlockSpec((_R, 128), lambda i: (i, 0)),
    out_shape=jax.ShapeDtypeStruct((_N, 128), jnp.float32),
)


def _tc_scale_body(deg0_ref, deg1_ref, m1_ref, m1p_ref, m1pb_ref, dinv_ref):
    deg = deg0_ref[...] + deg1_ref[...] + 1.0
    dinv = 1.0 / jnp.sqrt(deg)
    m1p = m1_ref[...] * dinv
    m1p_ref[...] = m1p
    m1pb_ref[...] = m1p.astype(jnp.bfloat16)
    dinv_ref[...] = dinv


_tc_scale = pl.pallas_call(
    _tc_scale_body,
    grid=(_G,),
    in_specs=[
        pl.BlockSpec((_R, 1), lambda i: (i, 0)),
        pl.BlockSpec((_R, 1), lambda i: (i, 0)),
        pl.BlockSpec((_R, 128), lambda i: (i, 0)),
    ],
    out_specs=[
        pl.BlockSpec((_R, 128), lambda i: (i, 0)),
        pl.BlockSpec((_R, 128), lambda i: (i, 0)),
        pl.BlockSpec((_R, 1), lambda i: (i, 0)),
    ],
    out_shape=[
        jax.ShapeDtypeStruct((_N, 128), jnp.float32),
        jax.ShapeDtypeStruct((_N, 128), jnp.bfloat16),
        jax.ShapeDtypeStruct((_N, 1), jnp.float32),
    ],
)


def _tcB_body(a0_ref, a1_ref, m1p_ref, dinv_ref, b1_ref, w2_ref,
              m2p_ref, m2pb_ref):
    dinv = dinv_ref[...]
    agg = a0_ref[...].astype(jnp.float32) + a1_ref[...].astype(jnp.float32)
    pre = dinv * (agg + m1p_ref[...]) + b1_ref[...]
    h1 = jnp.maximum(pre, 0.0)
    m2 = jnp.dot(h1, w2_ref[...], preferred_element_type=jnp.float32)
    m2p = m2 * dinv
    m2p_ref[...] = m2p
    m2pb_ref[...] = m2p.astype(jnp.bfloat16)


_tcB = pl.pallas_call(
    _tcB_body,
    grid=(_G,),
    in_specs=[
        pl.BlockSpec((_R, 128), lambda i: (i, 0)),
        pl.BlockSpec((_R, 128), lambda i: (i, 0)),
        pl.BlockSpec((_R, 128), lambda i: (i, 0)),
        pl.BlockSpec((_R, 1), lambda i: (i, 0)),
        pl.BlockSpec((1, 128), lambda i: (0, 0)),
        pl.BlockSpec((128, 64), lambda i: (0, 0)),
    ],
    out_specs=[
        pl.BlockSpec((_R, 64), lambda i: (i, 0)),
        pl.BlockSpec((_R, 64), lambda i: (i, 0)),
    ],
    out_shape=[
        jax.ShapeDtypeStruct((_N, 64), jnp.float32),
        jax.ShapeDtypeStruct((_N, 64), jnp.bfloat16),
    ],
)


def _tcC_body(a0_ref, a1_ref, m2p_ref, dinv_ref, b2_ref,
              fw1_ref, fb1_ref, fw2_ref, fb2_ref,
              asn_ref, pen_ref, s_ref):
    i = pl.program_id(0)
    agg = a0_ref[...].astype(jnp.float32) + a1_ref[...].astype(jnp.float32)
    h2 = dinv_ref[...] * (agg + m2p_ref[...]) + b2_ref[...]
    t = jnp.tanh(jnp.dot(h2, fw1_ref[...], preferred_element_type=jnp.float32)
                 + fb1_ref[...])
    logits = jnp.dot(t, fw2_ref[...], preferred_element_type=jnp.float32) + fb2_ref[...]
    mx = jnp.max(logits, axis=1, keepdims=True)
    e = jnp.exp(logits - mx)
    asn = e / jnp.sum(e, axis=1, keepdims=True)
    asn_ref[...] = asn
    d = asn - 0.5
    s1 = jnp.sum(d)
    s2 = jnp.sum(d * d)

    @pl.when(i == 0)
    def _():
        s_ref[0] = s1
        s_ref[1] = s2

    @pl.when(i > 0)
    def _():
        s_ref[0] += s1
        s_ref[1] += s2

    @pl.when(i == pl.num_programs(0) - 1)
    def _():
        n = 2.0 * _N
        var = (s_ref[1] - s_ref[0] * s_ref[0] / n) / (n - 1.0)
        pen_ref[...] = jnp.full((1, 1), var, dtype=jnp.float32)


_tcC = pl.pallas_call(
    _tcC_body,
    grid=(_G,),
    in_specs=[
        pl.BlockSpec((_R, 64), lambda i: (i, 0)),
        pl.BlockSpec((_R, 64), lambda i: (i, 0)),
        pl.BlockSpec((_R, 64), lambda i: (i, 0)),
        pl.BlockSpec((_R, 1), lambda i: (i, 0)),
        pl.BlockSpec((1, 64), lambda i: (0, 0)),
        pl.BlockSpec((64, 32), lambda i: (0, 0)),
        pl.BlockSpec((1, 32), lambda i: (0, 0)),
        pl.BlockSpec((32, 2), lambda i: (0, 0)),
        pl.BlockSpec((1, 2), lambda i: (0, 0)),
    ],
    out_specs=[
        pl.BlockSpec((_R, 2), lambda i: (i, 0)),
        pl.BlockSpec((1, 1), lambda i: (0, 0)),
    ],
    out_shape=[
        jax.ShapeDtypeStruct((_N, 2), jnp.float32),
        jax.ShapeDtypeStruct((1, 1), jnp.float32),
    ],
    scratch_shapes=[pltpu.SMEM((2,), jnp.float32)],
)


def kernel(x, edge_index, W1, b1, W2, b2, fc1_W, fc1_b, fc2_W, fc2_b):
    ei3 = edge_index.reshape(2, _E // _K, _K)
    ones_k = jnp.ones((_K,), jnp.float32)
    z1d = jnp.zeros((1024,), jnp.float32)
    z128 = jnp.zeros((_K, 128), jnp.bfloat16)
    z64 = jnp.zeros((_K, 64), jnp.bfloat16)

    m1 = _tc_mm(x, W1)
    degp = _deg_call(ei3, ones_k, z1d)
    deg0 = degp[:_N].reshape(_N, 1)
    deg1 = degp[_N:].reshape(_N, 1)
    m1p, m1pb, dinv = _tc_scale(deg0, deg1, m1)
    acc1 = _agg128(m1pb, ei3, z128)
    m2p, m2pb = _tcB(acc1[0], acc1[1], m1p, dinv, b1.reshape(1, -1), W2)
    acc2 = _agg64(m2pb, ei3, z64)
    asn, pen = _tcC(acc2[0], acc2[1], m2p, dinv, b2.reshape(1, -1),
                    fc1_W, fc1_b.reshape(1, -1), fc2_W, fc2_b.reshape(1, -1))
    return asn, pen.reshape(())


# speedup vs baseline: 43.3743x; 1.0504x over previous
import functools

import jax
import jax.numpy as jnp
from jax import lax
from jax.experimental import pallas as pl
from jax.experimental.pallas import tpu as pltpu
from jax.experimental.pallas import tpu_sc as plsc

_N = 10000
_E = 320000
_NC = 2
_NS = 16
_K = 80
_EPT = _E // (_NC * _NS)
_NBLK = _EPT // _K
_RCH = _N // _K
_ZJ = (_RCH + _NS - 1) // _NS


def _sc_mesh():
    return plsc.VectorSubcoreMesh(core_axis_name="c", subcore_axis_name="s")


def _deg_body(ei3_hbm, ones_hbm, z1d_hbm, out_hbm, didx_v, ones_v, stg_v,
              deg_sh, sem_a, sem_b):
    c = lax.axis_index("c")
    s = lax.axis_index("s")
    wid = c * _NS + s
    cp = pltpu.async_copy(ei3_hbm.at[1, pl.ds(wid * _NBLK, _NBLK)], didx_v,
                          sem_a)
    pltpu.sync_copy(ones_hbm, ones_v)
    pltpu.sync_copy(z1d_hbm.at[pl.ds(0, 640)], stg_v)
    pltpu.sync_copy(stg_v.at[pl.ds(0, 624)], deg_sh.at[pl.ds(s * 624, 624)])

    @pl.when(s == 0)
    def _():
        pltpu.sync_copy(stg_v.at[pl.ds(0, 16)], deg_sh.at[pl.ds(9984, 16)])

    cp.wait()
    plsc.subcore_barrier()

    def _fire(i, sem):
        pltpu.async_copy(ones_v, deg_sh.at[didx_v.at[i]], sem, add=True)

    def _drain(i, sem):
        pltpu.make_async_copy(ones_v, deg_sh.at[didx_v.at[i]], sem).wait()

    _fire(0, sem_a)

    def body(j, carry):
        i0 = 2 * j
        i1 = 2 * j + 1
        i2 = 2 * j + 2

        @pl.when(i1 < _NBLK)
        def _():
            _fire(i1, sem_b)

        _drain(i0, sem_a)

        @pl.when(i2 < _NBLK)
        def _():
            _fire(i2, sem_a)

        @pl.when(i1 < _NBLK)
        def _():
            _drain(i1, sem_b)

        return carry

    lax.fori_loop(0, (_NBLK + 1) // 2, body, 0)
    plsc.subcore_barrier()
    pltpu.sync_copy(deg_sh.at[pl.ds(s * 624, 624)], stg_v.at[pl.ds(0, 624)])
    pltpu.sync_copy(stg_v.at[pl.ds(0, 624)],
                    out_hbm.at[pl.ds(c * _N + s * 624, 624)])

    @pl.when(s == 0)
    def _():
        pltpu.sync_copy(deg_sh.at[pl.ds(9984, 16)], stg_v.at[pl.ds(624, 16)])
        pltpu.sync_copy(stg_v.at[pl.ds(624, 16)],
                        out_hbm.at[pl.ds(c * _N + 9984, 16)])


_deg_call = pl.kernel(
    _deg_body,
    out_type=jax.ShapeDtypeStruct((_NC * _N,), jnp.float32),
    mesh=_sc_mesh(),
    scratch_types=[
        pltpu.VMEM((_NBLK, _K), jnp.int32),
        pltpu.VMEM((_K,), jnp.float32),
        pltpu.VMEM((640,), jnp.float32),
        pltpu.VMEM_SHARED((_N,), jnp.float32),
        pltpu.SemaphoreType.DMA,
        pltpu.SemaphoreType.DMA,
    ],
    compiler_params=pltpu.CompilerParams(use_tc_tiling_on_sc=False),
)


_NBUF = 6
_JMAIN = _NBLK // _NBUF


def _agg_body(m_hbm, ei3_hbm, zrows_hbm, out_hbm,
              sidx_v, didx_v, r0, r1, r2, r3, r4, r5, acc_sh,
              g0, g1, g2, g3, g4, g5, s0, s1, s2, s3, s4, s5):
    rows = (r0, r1, r2, r3, r4, r5)
    gsem = (g0, g1, g2, g3, g4, g5)
    ssem = (s0, s1, s2, s3, s4, s5)
    c = lax.axis_index("c")
    s = lax.axis_index("s")
    wid = c * _NS + s

    cps = pltpu.async_copy(ei3_hbm.at[0, pl.ds(wid * _NBLK, _NBLK)], sidx_v,
                           g0)
    cpd = pltpu.async_copy(ei3_hbm.at[1, pl.ds(wid * _NBLK, _NBLK)], didx_v,
                           g1)
    pltpu.sync_copy(zrows_hbm, r0)

    def zbody(j, carry):
        ch = s + j * _NS

        @pl.when(ch < _RCH)
        def _():
            pltpu.sync_copy(r0, acc_sh.at[pl.ds(ch * _K, _K)])

        return carry

    lax.fori_loop(0, _ZJ, zbody, 0)
    cps.wait()
    cpd.wait()
    plsc.subcore_barrier()

    def _gstart(i, t):
        pltpu.async_copy(m_hbm.at[sidx_v.at[i]], rows[t], gsem[t])

    def _gwait(i, t):
        pltpu.make_async_copy(m_hbm.at[sidx_v.at[i]], rows[t], gsem[t]).wait()

    def _sstart(i, t):
        pltpu.async_copy(rows[t], acc_sh.at[didx_v.at[i]], ssem[t], add=True)

    def _swait(i, t):
        pltpu.make_async_copy(rows[t], acc_sh.at[didx_v.at[i]],
                              ssem[t]).wait()

    for t in range(_NBUF):
        _gstart(t, t)

    def ebody(j, carry):
        base = _NBUF * j
        for t in range(_NBUF):
            i = base + t
            _gwait(i, t)
            _sstart(i, t)
        for t in range(_NBUF):
            i = base + t

            @pl.when(i + _NBUF < _NBLK)
            def _():
                _swait(i, t)
                _gstart(i + _NBUF, t)

        return carry

    lax.fori_loop(0, _JMAIN, ebody, 0)
    for i in range(_JMAIN * _NBUF, _NBLK):
        _gwait(i, i % _NBUF)
        _sstart(i, i % _NBUF)
    for i in range(_NBLK - _NBUF, _NBLK):
        _swait(i, i % _NBUF)
    plsc.subcore_barrier()

    def obody(j, carry):
        ch = s + j * _NS

        @pl.when(ch < _RCH)
        def _():
            pltpu.sync_copy(acc_sh.at[pl.ds(ch * _K, _K)], r0)
            pltpu.sync_copy(r0, out_hbm.at[c, pl.ds(ch * _K, _K)])

        return carry

    lax.fori_loop(0, _ZJ, obody, 0)


def _make_agg(d):
    return pl.kernel(
        _agg_body,
        out_type=jax.ShapeDtypeStruct((_NC, _N, d), jnp.bfloat16),
        mesh=_sc_mesh(),
        scratch_types=[
            pltpu.VMEM((_NBLK, _K), jnp.int32),
            pltpu.VMEM((_NBLK, _K), jnp.int32),
            pltpu.VMEM((_K, d), jnp.bfloat16),
            pltpu.VMEM((_K, d), jnp.bfloat16),
            pltpu.VMEM((_K, d), jnp.bfloat16),
            pltpu.VMEM((_K, d), jnp.bfloat16),
            pltpu.VMEM((_K, d), jnp.bfloat16),
            pltpu.VMEM((_K, d), jnp.bfloat16),
            pltpu.VMEM_SHARED((_N, d), jnp.bfloat16),
            pltpu.SemaphoreType.DMA,
            pltpu.SemaphoreType.DMA,
            pltpu.SemaphoreType.DMA,
            pltpu.SemaphoreType.DMA,
            pltpu.SemaphoreType.DMA,
            pltpu.SemaphoreType.DMA,
            pltpu.SemaphoreType.DMA,
            pltpu.SemaphoreType.DMA,
            pltpu.SemaphoreType.DMA,
            pltpu.SemaphoreType.DMA,
            pltpu.SemaphoreType.DMA,
            pltpu.SemaphoreType.DMA,
        ],
        compiler_params=pltpu.CompilerParams(use_tc_tiling_on_sc=False),
    )


_agg128 = _make_agg(128)
_agg64 = _make_agg(64)


_R = 1000
_G = _N // _R


def _tc_mm_body(x_ref, w1_ref, m1_ref):
    m1_ref[...] = jnp.dot(x_ref[...], w1_ref[...],
                          preferred_element_type=jnp.float32)


_tc_mm = pl.pallas_call(
    _tc_mm_body,
    grid=(_G,),
    in_specs=[
        pl.BlockSpec((_R, 128), lambda i: (i, 0)),
        pl.BlockSpec((128, 128), lambda i: (0, 0)),
    ],
    out_specs=pl.BlockSpec((_R, 128), lambda i: (i, 0)),
    out_shape=jax.ShapeDtypeStruct((_N, 128), jnp.float32),
)


def _tc_scale_body(deg_ref, m1_ref, m1p_ref, m1pb_ref, dinv_ref):
    i = pl.program_id(0)
    d0 = deg_ref[pl.ds(i, 1), :]
    d1 = deg_ref[pl.ds(i + _G, 1), :]
    deg_row = d0 + d1 + 1.0
    dinv = 1.0 / jnp.sqrt(jnp.transpose(deg_row, (1, 0)))
    m1p = m1_ref[...] * dinv
    m1p_ref[...] = m1p
    m1pb_ref[...] = m1p.astype(jnp.bfloat16)
    dinv_ref[...] = dinv


_tc_scale = pl.pallas_call(
    _tc_scale_body,
    grid=(_G,),
    in_specs=[
        pl.BlockSpec((2 * _G, _R), lambda i: (0, 0)),
        pl.BlockSpec((_R, 128), lambda i: (i, 0)),
    ],
    out_specs=[
        pl.BlockSpec((_R, 128), lambda i: (i, 0)),
        pl.BlockSpec((_R, 128), lambda i: (i, 0)),
        pl.BlockSpec((_R, 1), lambda i: (i, 0)),
    ],
    out_shape=[
        jax.ShapeDtypeStruct((_N, 128), jnp.float32),
        jax.ShapeDtypeStruct((_N, 128), jnp.bfloat16),
        jax.ShapeDtypeStruct((_N, 1), jnp.float32),
    ],
)


def _tcB_body(a0_ref, a1_ref, m1p_ref, dinv_ref, b1_ref, w2_ref,
              m2p_ref, m2pb_ref):
    dinv = dinv_ref[...]
    agg = a0_ref[...].astype(jnp.float32) + a1_ref[...].astype(jnp.float32)
    pre = dinv * (agg + m1p_ref[...]) + b1_ref[...]
    h1 = jnp.maximum(pre, 0.0)
    m2 = jnp.dot(h1, w2_ref[...], preferred_element_type=jnp.float32)
    m2p = m2 * dinv
    m2p_ref[...] = m2p
    m2pb_ref[...] = m2p.astype(jnp.bfloat16)


_tcB = pl.pallas_call(
    _tcB_body,
    grid=(_G,),
    in_specs=[
        pl.BlockSpec((_R, 128), lambda i: (i, 0)),
        pl.BlockSpec((_R, 128), lambda i: (i, 0)),
        pl.BlockSpec((_R, 128), lambda i: (i, 0)),
        pl.BlockSpec((_R, 1), lambda i: (i, 0)),
        pl.BlockSpec((1, 128), lambda i: (0, 0)),
        pl.BlockSpec((128, 64), lambda i: (0, 0)),
    ],
    out_specs=[
        pl.BlockSpec((_R, 64), lambda i: (i, 0)),
        pl.BlockSpec((_R, 64), lambda i: (i, 0)),
    ],
    out_shape=[
        jax.ShapeDtypeStruct((_N, 64), jnp.float32),
        jax.ShapeDtypeStruct((_N, 64), jnp.bfloat16),
    ],
)


def _tcC_body(a0_ref, a1_ref, m2p_ref, dinv_ref, b2_ref,
              fw1_ref, fb1_ref, fw2_ref, fb2_ref,
              asn_ref, pen_ref, s_ref):
    i = pl.program_id(0)
    agg = a0_ref[...].astype(jnp.float32) + a1_ref[...].astype(jnp.float32)
    h2 = dinv_ref[...] * (agg + m2p_ref[...]) + b2_ref[...]
    t = jnp.tanh(jnp.dot(h2, fw1_ref[...], preferred_element_type=jnp.float32)
                 + fb1_ref[...])
    logits = jnp.dot(t, fw2_ref[...], preferred_element_type=jnp.float32) + fb2_ref[...]
    mx = jnp.max(logits, axis=1, keepdims=True)
    e = jnp.exp(logits - mx)
    asn = e / jnp.sum(e, axis=1, keepdims=True)
    asn_ref[...] = asn
    d = asn - 0.5
    s1 = jnp.sum(d)
    s2 = jnp.sum(d * d)

    @pl.when(i == 0)
    def _():
        s_ref[0] = s1
        s_ref[1] = s2

    @pl.when(i > 0)
    def _():
        s_ref[0] += s1
        s_ref[1] += s2

    @pl.when(i == pl.num_programs(0) - 1)
    def _():
        n = 2.0 * _N
        var = (s_ref[1] - s_ref[0] * s_ref[0] / n) / (n - 1.0)
        pen_ref[...] = jnp.full((1, 1), var, dtype=jnp.float32)


_tcC = pl.pallas_call(
    _tcC_body,
    grid=(_G,),
    in_specs=[
        pl.BlockSpec((_R, 64), lambda i: (i, 0)),
        pl.BlockSpec((_R, 64), lambda i: (i, 0)),
        pl.BlockSpec((_R, 64), lambda i: (i, 0)),
        pl.BlockSpec((_R, 1), lambda i: (i, 0)),
        pl.BlockSpec((1, 64), lambda i: (0, 0)),
        pl.BlockSpec((64, 32), lambda i: (0, 0)),
        pl.BlockSpec((1, 32), lambda i: (0, 0)),
        pl.BlockSpec((32, 2), lambda i: (0, 0)),
        pl.BlockSpec((1, 2), lambda i: (0, 0)),
    ],
    out_specs=[
        pl.BlockSpec((_R, 2), lambda i: (i, 0)),
        pl.BlockSpec((1, 1), lambda i: (0, 0)),
    ],
    out_shape=[
        jax.ShapeDtypeStruct((_N, 2), jnp.float32),
        jax.ShapeDtypeStruct((1, 1), jnp.float32),
    ],
    scratch_shapes=[pltpu.SMEM((2,), jnp.float32)],
)


def kernel(x, edge_index, W1, b1, W2, b2, fc1_W, fc1_b, fc2_W, fc2_b):
    ei3 = edge_index.reshape(2, _E // _K, _K)
    ones_k = jnp.ones((_K,), jnp.float32)
    z1d = jnp.zeros((1024,), jnp.float32)
    z128 = jnp.zeros((_K, 128), jnp.bfloat16)
    z64 = jnp.zeros((_K, 64), jnp.bfloat16)

    m1 = _tc_mm(x, W1)
    degp = _deg_call(ei3, ones_k, z1d)
    degr = degp.reshape(2 * _G, _R)
    m1p, m1pb, dinv = _tc_scale(degr, m1)
    acc1 = _agg128(m1pb, ei3, z128)
    m2p, m2pb = _tcB(acc1[0], acc1[1], m1p, dinv, b1.reshape(1, -1), W2)
    acc2 = _agg64(m2pb, ei3, z64)
    asn, pen = _tcC(acc2[0], acc2[1], m2p, dinv, b2.reshape(1, -1),
                    fc1_W, fc1_b.reshape(1, -1), fc2_W, fc2_b.reshape(1, -1))
    return asn, pen.reshape(())
